# Initial kernel scaffold; baseline (speedup 1.0000x reference)
import jax
import jax.numpy as jnp
from jax.experimental import pallas as pl


def kernel(x, edge_index, W_in, b_in, W1, b1, W2, b2, W3, b3, W_out, b_out, att_w, att_b, wp):
    raise NotImplementedError("write your pallas kernel here")


# speedup vs baseline: 1.0717x; 1.0717x over previous
import jax
import jax.numpy as jnp
from jax.experimental import pallas as pl

TAU = 0.9
K_HOPS = 2


def _identity_kernel(x_ref, o_ref):
    o_ref[...] = x_ref[...]


def kernel(x, edge_index, W_in, b_in, W1, b1, W2, b2, W3, b3, W_out, b_out, att_w, att_b, wp):
    num_nodes = x.shape[0]
    xh = jax.nn.relu(x @ W_in + b_in)
    xh = pl.pallas_call(
        _identity_kernel,
        out_shape=jax.ShapeDtypeStruct(xh.shape, xh.dtype),
    )(xh)
    src = edge_index[0]
    dst = edge_index[1]
    delta_x = jnp.abs(xh).sum(axis=1)
    neigh_sum = jnp.zeros((num_nodes,), jnp.float32).at[dst].add(delta_x[src])
    score = xh @ wp[:, 0] + neigh_sum
    pi = jax.nn.sigmoid(score)
    sel = pi >= TAU
    any_sel = jnp.any(sel)
    frontier = sel
    tot = sel
    for _ in range(K_HOPS):
        hits = jnp.zeros((num_nodes,), jnp.int32).at[src].add(frontier[dst].astype(jnp.int32))
        frontier = hits > 0
        tot = tot | frontier
    total = jnp.where(any_sel, tot, jnp.ones_like(tot))
    rank = jnp.cumsum(total.astype(jnp.int32)) - 1
    edge_mask = total[src] & total[dst]
    ks0 = jnp.where(edge_mask, rank[src], 0)
    kd0 = jnp.where(edge_mask, rank[dst], 0)
    valid = edge_mask & total[ks0] & total[kd0]
    ks = jnp.where(valid, ks0, 0)
    kd = jnp.where(valid, kd0, 0)
    h_i = xh[kd]
    h_j = xh[ks]
    p_j = pi[ks][:, None]
    e_ij = jnp.concatenate([h_i, h_j, p_j], axis=1)
    e = jax.nn.leaky_relu((e_ij @ att_w + att_b)[:, 0], 0.2)
    exp_e = jnp.where(valid, jnp.exp(e), 0.0)
    denom = jnp.zeros((num_nodes,), jnp.float32).at[kd].add(exp_e)
    alpha = jnp.where(valid, exp_e / (denom[kd] + 1e-16), 0.0)
    h = xh
    for W, b in ((W1, b1), (W2, b2), (W3, b3)):
        hx = h @ W + b
        msg = alpha[:, None] * hx[ks]
        h = jax.nn.relu(jnp.zeros_like(hx).at[kd].add(msg))
    logits = h @ W_out + b_out
    full_logits = jnp.where(total[:, None], logits, jnp.zeros_like(logits))
    full_logits = jnp.where(jnp.any(total), full_logits, jnp.zeros_like(full_logits))
    return full_logits


# speedup vs baseline: 10.1957x; 9.5139x over previous
import functools

import jax
import jax.numpy as jnp
from jax import lax
from jax.experimental import pallas as pl
from jax.experimental.pallas import tpu as pltpu
from jax.experimental.pallas import tpu_sc as plsc

N = 10000
E = 160000
NP = 10240
SL = 640
NT = 16
ET = E // NT
EC = 80
NCH = ET // EC
WAVE = 5
TAU = 0.9

_F32 = jnp.float32
_I32 = jnp.int32


def _dot(a, b):
    return jax.lax.dot_general(
        a, b, (((1,), (0,)), ((), ())),
        precision=jax.lax.Precision.HIGHEST,
        preferred_element_type=_F32)


def _prelude_body(x_ref, w_ref, b_ref, ws_ref, xhl_ref, xhr_ref, scal_ref):
    i = pl.program_id(0)
    xh = jnp.maximum(_dot(x_ref[...], w_ref[...]) + b_ref[0:1, :], 0.0)
    xhl_ref[...] = xh[:, :128]
    xhr_ref[...] = xh[:, 128:]
    s = _dot(xh, ws_ref[...])
    d = jnp.sum(jnp.abs(xh), axis=1, keepdims=True)
    col = jax.lax.broadcasted_iota(_I32, (1024, 128), 1)
    row = i * 1024 + jax.lax.broadcasted_iota(_I32, (1024, 128), 0)
    s = s + jnp.where(col == 3, d, 0.0)
    scal_ref[...] = jnp.where(row < N, s, -1e9)


def _prelude(xp, W_in, b_in, Ws):
    return pl.pallas_call(
        _prelude_body,
        grid=(10,),
        in_specs=[
            pl.BlockSpec((1024, 256), lambda i: (i, 0)),
            pl.BlockSpec((256, 256), lambda i: (0, 0)),
            pl.BlockSpec((8, 256), lambda i: (0, 0)),
            pl.BlockSpec((256, 128), lambda i: (0, 0)),
        ],
        out_specs=[
            pl.BlockSpec((1024, 128), lambda i: (i, 0)),
            pl.BlockSpec((1024, 128), lambda i: (i, 0)),
            pl.BlockSpec((1024, 128), lambda i: (i, 0)),
        ],
        out_shape=[
            jax.ShapeDtypeStruct((NP, 128), _F32),
            jax.ShapeDtypeStruct((NP, 128), _F32),
            jax.ShapeDtypeStruct((NP, 128), _F32),
        ],
    )(xp, W_in, b_in, Ws)


def _layer_body(hl_ref, hr_ref, inv_ref, w_ref, b_ref, ol_ref, or_ref):
    h = jnp.concatenate([hl_ref[...], hr_ref[...]], axis=1)
    h = jnp.maximum(h * inv_ref[...], 0.0)
    hx = _dot(h, w_ref[...]) + b_ref[0:1, :]
    ol_ref[...] = hx[:, :128]
    or_ref[...] = hx[:, 128:]


def _layer(hl, hr, inv, W, b):
    return pl.pallas_call(
        _layer_body,
        grid=(10,),
        in_specs=[
            pl.BlockSpec((1024, 128), lambda i: (i, 0)),
            pl.BlockSpec((1024, 128), lambda i: (i, 0)),
            pl.BlockSpec((1024, 1), lambda i: (i, 0)),
            pl.BlockSpec((256, 256), lambda i: (0, 0)),
            pl.BlockSpec((8, 256), lambda i: (0, 0)),
        ],
        out_specs=[
            pl.BlockSpec((1024, 128), lambda i: (i, 0)),
            pl.BlockSpec((1024, 128), lambda i: (i, 0)),
        ],
        out_shape=[
            jax.ShapeDtypeStruct((NP, 128), _F32),
            jax.ShapeDtypeStruct((NP, 128), _F32),
        ],
    )(hl, hr, inv, W, b)


def _final_body(hl_ref, hr_ref, inv_ref, w_ref, b_ref, m_ref, o_ref):
    h = jnp.concatenate([hl_ref[...], hr_ref[...]], axis=1)
    h = jnp.maximum(h * inv_ref[...], 0.0)
    o_ref[...] = (_dot(h, w_ref[...]) + b_ref[0:1, :]) * m_ref[...]


def _final(hl, hr, inv, W, b, mask):
    return pl.pallas_call(
        _final_body,
        grid=(10,),
        in_specs=[
            pl.BlockSpec((1024, 128), lambda i: (i, 0)),
            pl.BlockSpec((1024, 128), lambda i: (i, 0)),
            pl.BlockSpec((1024, 1), lambda i: (i, 0)),
            pl.BlockSpec((256, 128), lambda i: (0, 0)),
            pl.BlockSpec((8, 128), lambda i: (0, 0)),
            pl.BlockSpec((1024, 1), lambda i: (i, 0)),
        ],
        out_specs=pl.BlockSpec((1024, 128), lambda i: (i, 0)),
        out_shape=jax.ShapeDtypeStruct((NP, 128), _F32),
    )(hl, hr, inv, W, b, mask)


def _k1_body(src_h, dst_h, delta_h, sp_h, si_h, sj_h, ap_h,
             ks_o, kd_o, ex_o, inv_o, tot_o,
             src2, dst2, nodef, sjf, toti, ranki,
             ksb, kdb, exb, sp_s, tmp, tmpi, tots, zbf, s16, flat256, apv,
             accA, shf, shi, sh16, sem):
    t = lax.axis_index("s")
    sl = pl.ds(t * SL, SL)
    ZV = jnp.zeros((16,), _F32)
    ZI = jnp.zeros((16,), _I32)

    def share_i(slice_ref, full_ref):
        pltpu.sync_copy(slice_ref, shi.at[sl])
        plsc.subcore_barrier()
        pltpu.sync_copy(shi, full_ref)
        plsc.subcore_barrier()

    def share_f(slice_ref, full_ref):
        pltpu.sync_copy(slice_ref, shf.at[sl])
        plsc.subcore_barrier()
        pltpu.sync_copy(shf, full_ref)
        plsc.subcore_barrier()

    def fire_adds(val2, idx2, acc):
        for w in range(NCH // WAVE):
            ds_ = [pltpu.async_copy(val2.at[w * WAVE + b],
                                    acc.at[idx2.at[w * WAVE + b]],
                                    sem, add=True)
                   for b in range(WAVE)]
            for dd in ds_:
                dd.wait()

    pltpu.sync_copy(src_h.at[t], src2)
    pltpu.sync_copy(dst_h.at[t], dst2)
    pltpu.sync_copy(delta_h, nodef)
    pltpu.sync_copy(sp_h.at[sl], sp_s)
    pltpu.sync_copy(sj_h, sjf)
    pltpu.sync_copy(ap_h, apv)

    def zf(j, _):
        tmp[pl.ds(j * 16, 16)] = ZV
        zbf[pl.ds(j * 16, 16)] = ZV
        return 0
    lax.fori_loop(0, SL // 16, zf, 0)
    pltpu.sync_copy(tmp, accA.at[sl])
    plsc.subcore_barrier()

    def av(c, _):
        def iv(j, _):
            idx = pl.ds(j * 16, 16)
            exb[c, idx] = plsc.load_gather(nodef, [src2[c, idx]])
            return 0
        lax.fori_loop(0, EC // 16, iv, 0)
        return 0
    lax.fori_loop(0, NCH, av, 0)
    fire_adds(exb, dst2, accA)
    plsc.subcore_barrier()
    pltpu.sync_copy(accA.at[sl], tmp)
    pltpu.sync_copy(zbf, accA.at[sl])

    def pv(j, anyv):
        idx = pl.ds(j * 16, 16)
        x = sp_s[idx] + tmp[idx]
        p = 1.0 / (1.0 + jnp.exp(-x))
        sp_s[idx] = p
        selv = (p >= TAU).astype(_I32)
        tots[idx] = selv
        tmpi[idx] = selv
        return jnp.maximum(anyv, selv)
    anyv = lax.fori_loop(0, SL // 16, pv, ZI)
    s16[pl.ds(0, 16)] = anyv
    pltpu.sync_copy(s16, sh16.at[pl.ds(t * 16, 16)])
    share_i(tmpi, ranki)
    pltpu.sync_copy(sh16, flat256)

    def rmax(k, a):
        return jnp.maximum(a, flat256[pl.ds(k * 16, 16)])
    any_s = jnp.max(lax.fori_loop(0, 16, rmax, ZI))

    for _hop in range(2):
        def bv(c, _):
            def ivb(j, _):
                idx = pl.ds(j * 16, 16)
                exb[c, idx] = plsc.load_gather(
                    ranki, [dst2[c, idx]]).astype(_F32)
                return 0
            lax.fori_loop(0, EC // 16, ivb, 0)
            return 0
        lax.fori_loop(0, NCH, bv, 0)
        fire_adds(exb, src2, accA)
        plsc.subcore_barrier()
        pltpu.sync_copy(accA.at[sl], tmp)
        pltpu.sync_copy(zbf, accA.at[sl])

        def fv(j, _):
            idx = pl.ds(j * 16, 16)
            fn = (tmp[idx] > 0.0).astype(_I32)
            tots[idx] = tots[idx] | fn
            tmpi[idx] = fn
            return 0
        lax.fori_loop(0, SL // 16, fv, 0)
        share_i(tmpi, ranki)

    def tv(j, cnt):
        idx = pl.ds(j * 16, 16)
        v = jnp.where(any_s > 0, tots[idx], 1)
        tots[idx] = v
        return cnt + jnp.sum(v)
    cnt = lax.fori_loop(0, SL // 16, tv, jnp.int32(0))
    s16[pl.ds(0, 16)] = jnp.full((16,), cnt, _I32)
    pltpu.sync_copy(s16, sh16.at[pl.ds(t * 16, 16)])
    share_i(tots, toti)
    pltpu.sync_copy(sh16, flat256)
    lanes = lax.iota(_I32, 16)
    cnts = plsc.load_gather(flat256, [lanes * 16])
    off = jnp.sum(jnp.where(lanes < t, cnts, 0))

    def rv(j, carry):
        idx = pl.ds(j * 16, 16)
        v = tots[idx]
        tmpi[idx] = plsc.cumsum(v) + (off + carry - 1)
        return carry + jnp.sum(v)
    lax.fori_loop(0, SL // 16, rv, jnp.int32(0))
    share_i(tmpi, ranki)

    share_f(sp_s, nodef)
    apl = apv[pl.ds(0, 16)]

    def tj(j, _):
        idx = pl.ds(j * 16, 16)
        sjf[idx] = sjf[idx] + apl * nodef[idx]
        return 0
    lax.fori_loop(0, NP // 16, tj, 0)
    pltpu.sync_copy(si_h, nodef)

    def dvl(c, _):
        def ivd(j, _):
            idx = pl.ds(j * 16, 16)
            sv = src2[c, idx]
            dv = dst2[c, idx]
            ts = plsc.load_gather(toti, [sv])
            td = plsc.load_gather(toti, [dv])
            em = ts & td
            rs = plsc.load_gather(ranki, [sv])
            rd = plsc.load_gather(ranki, [dv])
            emb = em > 0
            ks0 = jnp.where(emb, rs, 0)
            kd0 = jnp.where(emb, rd, 0)
            tks = plsc.load_gather(toti, [ks0])
            tkd = plsc.load_gather(toti, [kd0])
            vab = (em & tks & tkd) > 0
            ksv = jnp.where(vab, ks0, 0)
            kdv = jnp.where(vab, kd0, 0)
            e = plsc.load_gather(nodef, [kdv]) + plsc.load_gather(sjf, [ksv])
            e = jnp.maximum(e, 0.2 * e)
            exv = jnp.where(vab, jnp.exp(e), 0.0)
            ksb[c, idx] = ksv
            kdb[c, idx] = kdv
            exb[c, idx] = exv
            return 0
        lax.fori_loop(0, EC // 16, ivd, 0)
        return 0
    lax.fori_loop(0, NCH, dvl, 0)
    fire_adds(exb, kdb, accA)
    plsc.subcore_barrier()

    pltpu.sync_copy(ksb, ks_o.at[t])
    pltpu.sync_copy(kdb, kd_o.at[t])
    pltpu.sync_copy(exb, ex_o.at[t])
    pltpu.sync_copy(accA.at[sl], tmp)

    def iv2(j, _):
        idx = pl.ds(j * 16, 16)
        tmp[idx] = 1.0 / (tmp[idx] + 1e-16)
        return 0
    lax.fori_loop(0, SL // 16, iv2, 0)
    pltpu.sync_copy(tmp, inv_o.at[sl])

    def tf2(j, _):
        idx = pl.ds(j * 16, 16)
        tmp[idx] = tots[idx].astype(_F32)
        return 0
    lax.fori_loop(0, SL // 16, tf2, 0)
    pltpu.sync_copy(tmp, tot_o.at[sl])


def _edge_prep(src_r, dst_r, delta, sp, si, sj, apv):
    k1 = pl.kernel(
        _k1_body,
        out_type=[
            jax.ShapeDtypeStruct((NT, NCH, EC), _I32),
            jax.ShapeDtypeStruct((NT, NCH, EC), _I32),
            jax.ShapeDtypeStruct((NT, NCH, EC), _F32),
            jax.ShapeDtypeStruct((NP,), _F32),
            jax.ShapeDtypeStruct((NP,), _F32),
        ],
        mesh=plsc.VectorSubcoreMesh(
            core_axis_name="c", subcore_axis_name="s",
            num_cores=1, num_subcores=NT),
        scratch_types=[
            pltpu.VMEM((NCH, EC), _I32),
            pltpu.VMEM((NCH, EC), _I32),
            pltpu.VMEM((NP,), _F32),
            pltpu.VMEM((NP,), _F32),
            pltpu.VMEM((NP,), _I32),
            pltpu.VMEM((NP,), _I32),
            pltpu.VMEM((NCH, EC), _I32),
            pltpu.VMEM((NCH, EC), _I32),
            pltpu.VMEM((NCH, EC), _F32),
            pltpu.VMEM((SL,), _F32),
            pltpu.VMEM((SL,), _F32),
            pltpu.VMEM((SL,), _I32),
            pltpu.VMEM((SL,), _I32),
            pltpu.VMEM((SL,), _F32),
            pltpu.VMEM((16,), _I32),
            pltpu.VMEM((256,), _I32),
            pltpu.VMEM((16,), _F32),
            pltpu.VMEM_SHARED((NP,), _F32),
            pltpu.VMEM_SHARED((NP,), _F32),
            pltpu.VMEM_SHARED((NP,), _I32),
            pltpu.VMEM_SHARED((256,), _I32),
            pltpu.SemaphoreType.DMA,
        ],
        compiler_params=pltpu.CompilerParams(needs_layout_passes=False),
    )
    return k1(src_r, dst_r, delta, sp, si, sj, apv)


def _conv_body(hxl_h, hxr_h, ks_h, kd_h, ex_h, aggl_h, aggr_h,
               ks80, kd80, ex80, rows, acc, sem):
    cid = lax.axis_index("c")
    s = lax.axis_index("s")
    ZV = jnp.zeros((16,), _F32)

    def work(hx_h, agg_h):
        def zr(r, _):
            for k in range(8):
                rows[r, pl.ds(k * 16, 16)] = ZV
            return 0
        lax.fori_loop(0, EC, zr, 0)
        for k in range(SL // EC):
            pltpu.sync_copy(rows, acc.at[pl.ds(s * SL + k * EC, EC)])
        plsc.subcore_barrier()

        def chunk(c, _):
            base = s * ET + c * EC
            pltpu.sync_copy(ks_h.at[pl.ds(base, EC)], ks80)
            pltpu.sync_copy(kd_h.at[pl.ds(base, EC)], kd80)
            pltpu.sync_copy(ex_h.at[pl.ds(base, EC)], ex80)
            pltpu.async_copy(hx_h.at[ks80], rows, sem).wait()

            def rowf(r, _):
                wv = plsc.load_gather(ex80, [jnp.full((16,), r, _I32)])
                for k in range(8):
                    idx = pl.ds(k * 16, 16)
                    rows[r, idx] = rows[r, idx] * wv
                return 0
            lax.fori_loop(0, EC, rowf, 0)
            pltpu.sync_copy(rows, acc.at[kd80], add=True)
            return 0
        lax.fori_loop(0, NCH, chunk, 0)
        plsc.subcore_barrier()
        pltpu.sync_copy(acc.at[pl.ds(s * SL, SL)], agg_h.at[pl.ds(s * SL, SL)])

    @pl.when(cid == 0)
    def _():
        work(hxl_h, aggl_h)

    @pl.when(cid == 1)
    def _():
        work(hxr_h, aggr_h)


def _conv(hxl, hxr, ks, kd, ex):
    k2 = pl.kernel(
        _conv_body,
        out_type=[
            jax.ShapeDtypeStruct((NP, 128), _F32),
            jax.ShapeDtypeStruct((NP, 128), _F32),
        ],
        mesh=plsc.VectorSubcoreMesh(
            core_axis_name="c", subcore_axis_name="s",
            num_cores=2, num_subcores=NT),
        scratch_types=[
            pltpu.VMEM((EC,), _I32),
            pltpu.VMEM((EC,), _I32),
            pltpu.VMEM((EC,), _F32),
            pltpu.VMEM((EC, 128), _F32),
            pltpu.VMEM_SHARED((NP, 128), _F32),
            pltpu.SemaphoreType.DMA,
        ],
        compiler_params=pltpu.CompilerParams(needs_layout_passes=False),
    )
    return k2(hxl, hxr, ks, kd, ex)


def kernel(x, edge_index, W_in, b_in, W1, b1, W2, b2, W3, b3,
           W_out, b_out, att_w, att_b, wp):
    xp = jnp.pad(x, ((0, NP - N), (0, 0)))
    src_r = edge_index[0].reshape(NT, NCH, EC)
    dst_r = edge_index[1].reshape(NT, NCH, EC)
    Ws = jnp.pad(jnp.concatenate([wp, att_w[0:256], att_w[256:512]], axis=1),
                 ((0, 0), (0, 125)))
    apv = jnp.full((16,), att_w[512, 0], _F32)

    def b8(b):
        return jnp.broadcast_to(b[None, :], (8, b.shape[0]))

    xhl, xhr, scal = _prelude(xp, W_in, b8(b_in), Ws)
    sp = scal[:, 0]
    si = scal[:, 1] + att_b[0]
    sj = scal[:, 2]
    delta = scal[:, 3]

    ks, kd, ex, inv, totf = _edge_prep(src_r, dst_r, delta, sp, si, sj, apv)
    ks = ks.reshape(E)
    kd = kd.reshape(E)
    ex = ex.reshape(E)

    inv2 = inv.reshape(NP, 1)
    ones = jnp.ones((NP, 1), _F32)
    h1l, h1r = _layer(xhl, xhr, ones, W1, b8(b1))
    a1l, a1r = _conv(h1l, h1r, ks, kd, ex)
    h2l, h2r = _layer(a1l, a1r, inv2, W2, b8(b2))
    a2l, a2r = _conv(h2l, h2r, ks, kd, ex)
    h3l, h3r = _layer(a2l, a2r, inv2, W3, b8(b3))
    a3l, a3r = _conv(h3l, h3r, ks, kd, ex)
    out = _final(a3l, a3r, inv2, W_out, b8(b_out), totf.reshape(NP, 1))
    return out[:N]


# speedup vs baseline: 21.3690x; 2.0959x over previous
import functools

import jax
import jax.numpy as jnp
from jax import lax
from jax.experimental import pallas as pl
from jax.experimental.pallas import tpu as pltpu
from jax.experimental.pallas import tpu_sc as plsc

N = 10000
E = 160000
NP = 10240
SL = 640
NT = 16
ET = E // NT
EC = 80
NCH = ET // EC
WAVE = 5
TAU = 0.9

_F32 = jnp.float32
_I32 = jnp.int32


def _dot(a, b):
    return jax.lax.dot_general(
        a, b, (((1,), (0,)), ((), ())),
        precision=jax.lax.Precision.HIGHEST,
        preferred_element_type=_F32)


def _prelude_body(x_ref, w_ref, b_ref, ws_ref, xhl_ref, xhr_ref, scal_ref):
    i = pl.program_id(0)
    xh = jnp.maximum(_dot(x_ref[...], w_ref[...]) + b_ref[0:1, :], 0.0)
    xhl_ref[...] = xh[:, :128]
    xhr_ref[...] = xh[:, 128:]
    s = _dot(xh, ws_ref[...])
    d = jnp.sum(jnp.abs(xh), axis=1, keepdims=True)
    col = jax.lax.broadcasted_iota(_I32, (1024, 128), 1)
    row = i * 1024 + jax.lax.broadcasted_iota(_I32, (1024, 128), 0)
    s = s + jnp.where(col == 3, d, 0.0)
    scal_ref[...] = jnp.where(row < N, s, -1e9)


def _prelude(xp, W_in, b_in, Ws):
    return pl.pallas_call(
        _prelude_body,
        grid=(10,),
        in_specs=[
            pl.BlockSpec((1024, 256), lambda i: (i, 0)),
            pl.BlockSpec((256, 256), lambda i: (0, 0)),
            pl.BlockSpec((8, 256), lambda i: (0, 0)),
            pl.BlockSpec((256, 128), lambda i: (0, 0)),
        ],
        out_specs=[
            pl.BlockSpec((1024, 128), lambda i: (i, 0)),
            pl.BlockSpec((1024, 128), lambda i: (i, 0)),
            pl.BlockSpec((1024, 128), lambda i: (i, 0)),
        ],
        out_shape=[
            jax.ShapeDtypeStruct((NP, 128), _F32),
            jax.ShapeDtypeStruct((NP, 128), _F32),
            jax.ShapeDtypeStruct((NP, 128), _F32),
        ],
    )(xp, W_in, b_in, Ws)


def _layer_body(hl_ref, hr_ref, inv_ref, w_ref, b_ref, ol_ref, or_ref):
    h = jnp.concatenate([hl_ref[...], hr_ref[...]], axis=1)
    h = jnp.maximum(h * inv_ref[...], 0.0)
    hx = _dot(h, w_ref[...]) + b_ref[0:1, :]
    ol_ref[...] = hx[:, :128]
    or_ref[...] = hx[:, 128:]


def _layer(hl, hr, inv, W, b):
    return pl.pallas_call(
        _layer_body,
        grid=(10,),
        in_specs=[
            pl.BlockSpec((1024, 128), lambda i: (i, 0)),
            pl.BlockSpec((1024, 128), lambda i: (i, 0)),
            pl.BlockSpec((1024, 1), lambda i: (i, 0)),
            pl.BlockSpec((256, 256), lambda i: (0, 0)),
            pl.BlockSpec((8, 256), lambda i: (0, 0)),
        ],
        out_specs=[
            pl.BlockSpec((1024, 128), lambda i: (i, 0)),
            pl.BlockSpec((1024, 128), lambda i: (i, 0)),
        ],
        out_shape=[
            jax.ShapeDtypeStruct((NP, 128), _F32),
            jax.ShapeDtypeStruct((NP, 128), _F32),
        ],
    )(hl, hr, inv, W, b)


def _final_body(hl_ref, hr_ref, inv_ref, w_ref, b_ref, m_ref, o_ref):
    h = jnp.concatenate([hl_ref[...], hr_ref[...]], axis=1)
    h = jnp.maximum(h * inv_ref[...], 0.0)
    o_ref[...] = (_dot(h, w_ref[...]) + b_ref[0:1, :]) * m_ref[...]


def _final(hl, hr, inv, W, b, mask):
    return pl.pallas_call(
        _final_body,
        grid=(10,),
        in_specs=[
            pl.BlockSpec((1024, 128), lambda i: (i, 0)),
            pl.BlockSpec((1024, 128), lambda i: (i, 0)),
            pl.BlockSpec((1024, 1), lambda i: (i, 0)),
            pl.BlockSpec((256, 128), lambda i: (0, 0)),
            pl.BlockSpec((8, 128), lambda i: (0, 0)),
            pl.BlockSpec((1024, 1), lambda i: (i, 0)),
        ],
        out_specs=pl.BlockSpec((1024, 128), lambda i: (i, 0)),
        out_shape=jax.ShapeDtypeStruct((NP, 128), _F32),
    )(hl, hr, inv, W, b, mask)


def _k1_body(src_h, dst_h, delta_h, sp_h, si_h, sj_h, ap_h,
             ks_o, kd_o, ex_o, inv_o, tot_o,
             src2, dst2, nodef, sjf, toti, ranki,
             ksb, kdb, exb, sp_s, tmp, tmpi, tots, zbf, s16, flat256, apv,
             accA, shf, shi, sh16, sem):
    t = lax.axis_index("s")
    sl = pl.ds(t * SL, SL)
    ZV = jnp.zeros((16,), _F32)
    ZI = jnp.zeros((16,), _I32)

    def share_i(slice_ref, full_ref):
        pltpu.sync_copy(slice_ref, shi.at[sl])
        plsc.subcore_barrier()
        pltpu.sync_copy(shi, full_ref)
        plsc.subcore_barrier()

    def share_f(slice_ref, full_ref):
        pltpu.sync_copy(slice_ref, shf.at[sl])
        plsc.subcore_barrier()
        pltpu.sync_copy(shf, full_ref)
        plsc.subcore_barrier()

    def fire_adds(val2, idx2, acc):
        for w in range(NCH // WAVE):
            ds_ = [pltpu.async_copy(val2.at[w * WAVE + b],
                                    acc.at[idx2.at[w * WAVE + b]],
                                    sem, add=True)
                   for b in range(WAVE)]
            for dd in ds_:
                dd.wait()

    pltpu.sync_copy(src_h.at[t], src2)
    pltpu.sync_copy(dst_h.at[t], dst2)
    pltpu.sync_copy(delta_h, nodef)
    pltpu.sync_copy(sp_h.at[sl], sp_s)
    pltpu.sync_copy(sj_h, sjf)
    pltpu.sync_copy(ap_h, apv)

    def zf(j, _):
        tmp[pl.ds(j * 16, 16)] = ZV
        zbf[pl.ds(j * 16, 16)] = ZV
        return 0
    lax.fori_loop(0, SL // 16, zf, 0)
    pltpu.sync_copy(tmp, accA.at[sl])
    plsc.subcore_barrier()

    def av(c, _):
        def iv(j, _):
            idx = pl.ds(j * 16, 16)
            exb[c, idx] = plsc.load_gather(nodef, [src2[c, idx]])
            return 0
        lax.fori_loop(0, EC // 16, iv, 0)
        return 0
    lax.fori_loop(0, NCH, av, 0)
    fire_adds(exb, dst2, accA)
    plsc.subcore_barrier()
    pltpu.sync_copy(accA.at[sl], tmp)
    pltpu.sync_copy(zbf, accA.at[sl])

    def pv(j, anyv):
        idx = pl.ds(j * 16, 16)
        x = sp_s[idx] + tmp[idx]
        p = 1.0 / (1.0 + jnp.exp(-x))
        sp_s[idx] = p
        selv = (p >= TAU).astype(_I32)
        tots[idx] = selv
        tmpi[idx] = selv
        return jnp.maximum(anyv, selv)
    anyv = lax.fori_loop(0, SL // 16, pv, ZI)
    s16[pl.ds(0, 16)] = anyv
    pltpu.sync_copy(s16, sh16.at[pl.ds(t * 16, 16)])
    share_i(tmpi, ranki)
    pltpu.sync_copy(sh16, flat256)

    def rmax(k, a):
        return jnp.maximum(a, flat256[pl.ds(k * 16, 16)])
    any_s = jnp.max(lax.fori_loop(0, 16, rmax, ZI))

    for _hop in range(2):
        def bv(c, _):
            def ivb(j, _):
                idx = pl.ds(j * 16, 16)
                exb[c, idx] = plsc.load_gather(
                    ranki, [dst2[c, idx]]).astype(_F32)
                return 0
            lax.fori_loop(0, EC // 16, ivb, 0)
            return 0
        lax.fori_loop(0, NCH, bv, 0)
        fire_adds(exb, src2, accA)
        plsc.subcore_barrier()
        pltpu.sync_copy(accA.at[sl], tmp)
        pltpu.sync_copy(zbf, accA.at[sl])

        def fv(j, _):
            idx = pl.ds(j * 16, 16)
            fn = (tmp[idx] > 0.0).astype(_I32)
            tots[idx] = tots[idx] | fn
            tmpi[idx] = fn
            return 0
        lax.fori_loop(0, SL // 16, fv, 0)
        share_i(tmpi, ranki)

    def tv(j, cnt):
        idx = pl.ds(j * 16, 16)
        v = jnp.where(any_s > 0, tots[idx], 1)
        tots[idx] = v
        return cnt + jnp.sum(v)
    cnt = lax.fori_loop(0, SL // 16, tv, jnp.int32(0))
    s16[pl.ds(0, 16)] = jnp.full((16,), cnt, _I32)
    pltpu.sync_copy(s16, sh16.at[pl.ds(t * 16, 16)])
    share_i(tots, toti)
    pltpu.sync_copy(sh16, flat256)
    lanes = lax.iota(_I32, 16)
    cnts = plsc.load_gather(flat256, [lanes * 16])
    off = jnp.sum(jnp.where(lanes < t, cnts, 0))

    def rv(j, carry):
        idx = pl.ds(j * 16, 16)
        v = tots[idx]
        tmpi[idx] = plsc.cumsum(v) + (off + carry - 1)
        return carry + jnp.sum(v)
    lax.fori_loop(0, SL // 16, rv, jnp.int32(0))
    share_i(tmpi, ranki)

    share_f(sp_s, nodef)
    apl = apv[pl.ds(0, 16)]

    def tj(j, _):
        idx = pl.ds(j * 16, 16)
        sjf[idx] = sjf[idx] + apl * nodef[idx]
        return 0
    lax.fori_loop(0, NP // 16, tj, 0)
    pltpu.sync_copy(si_h, nodef)

    def dvl(c, _):
        def ivd(j, _):
            idx = pl.ds(j * 16, 16)
            sv = src2[c, idx]
            dv = dst2[c, idx]
            ts = plsc.load_gather(toti, [sv])
            td = plsc.load_gather(toti, [dv])
            em = ts & td
            rs = plsc.load_gather(ranki, [sv])
            rd = plsc.load_gather(ranki, [dv])
            emb = em > 0
            ks0 = jnp.where(emb, rs, 0)
            kd0 = jnp.where(emb, rd, 0)
            tks = plsc.load_gather(toti, [ks0])
            tkd = plsc.load_gather(toti, [kd0])
            vab = (em & tks & tkd) > 0
            ksv = jnp.where(vab, ks0, 0)
            kdv = jnp.where(vab, kd0, 0)
            e = plsc.load_gather(nodef, [kdv]) + plsc.load_gather(sjf, [ksv])
            e = jnp.maximum(e, 0.2 * e)
            exv = jnp.where(vab, jnp.exp(e), 0.0)
            ksb[c, idx] = ksv
            kdb[c, idx] = kdv
            exb[c, idx] = exv
            return 0
        lax.fori_loop(0, EC // 16, ivd, 0)
        return 0
    lax.fori_loop(0, NCH, dvl, 0)
    fire_adds(exb, kdb, accA)
    plsc.subcore_barrier()

    pltpu.sync_copy(ksb, ks_o.at[t])
    pltpu.sync_copy(kdb, kd_o.at[t])
    pltpu.sync_copy(exb, ex_o.at[t])
    pltpu.sync_copy(accA.at[sl], tmp)

    def iv2(j, _):
        idx = pl.ds(j * 16, 16)
        tmp[idx] = 1.0 / (tmp[idx] + 1e-16)
        return 0
    lax.fori_loop(0, SL // 16, iv2, 0)
    pltpu.sync_copy(tmp, inv_o.at[sl])

    def tf2(j, _):
        idx = pl.ds(j * 16, 16)
        tmp[idx] = tots[idx].astype(_F32)
        return 0
    lax.fori_loop(0, SL // 16, tf2, 0)
    pltpu.sync_copy(tmp, tot_o.at[sl])


def _edge_prep(src_r, dst_r, delta, sp, si, sj, apv):
    k1 = pl.kernel(
        _k1_body,
        out_type=[
            jax.ShapeDtypeStruct((NT, NCH, EC), _I32),
            jax.ShapeDtypeStruct((NT, NCH, EC), _I32),
            jax.ShapeDtypeStruct((NT, NCH, EC), _F32),
            jax.ShapeDtypeStruct((NP,), _F32),
            jax.ShapeDtypeStruct((NP,), _F32),
        ],
        mesh=plsc.VectorSubcoreMesh(
            core_axis_name="c", subcore_axis_name="s",
            num_cores=1, num_subcores=NT),
        scratch_types=[
            pltpu.VMEM((NCH, EC), _I32),
            pltpu.VMEM((NCH, EC), _I32),
            pltpu.VMEM((NP,), _F32),
            pltpu.VMEM((NP,), _F32),
            pltpu.VMEM((NP,), _I32),
            pltpu.VMEM((NP,), _I32),
            pltpu.VMEM((NCH, EC), _I32),
            pltpu.VMEM((NCH, EC), _I32),
            pltpu.VMEM((NCH, EC), _F32),
            pltpu.VMEM((SL,), _F32),
            pltpu.VMEM((SL,), _F32),
            pltpu.VMEM((SL,), _I32),
            pltpu.VMEM((SL,), _I32),
            pltpu.VMEM((SL,), _F32),
            pltpu.VMEM((16,), _I32),
            pltpu.VMEM((256,), _I32),
            pltpu.VMEM((16,), _F32),
            pltpu.VMEM_SHARED((NP,), _F32),
            pltpu.VMEM_SHARED((NP,), _F32),
            pltpu.VMEM_SHARED((NP,), _I32),
            pltpu.VMEM_SHARED((256,), _I32),
            pltpu.SemaphoreType.DMA,
        ],
        compiler_params=pltpu.CompilerParams(needs_layout_passes=False),
    )
    return k1(src_r, dst_r, delta, sp, si, sj, apv)


def _conv_body(hxl_h, hxr_h, ks_h, kd_h, ex_h, aggl_h, aggr_h,
               ks0, kd0, ex0, ks1, kd1, ex1, kc0, kc1, rows0, rows1, acc,
               sg0, sg1, ss0, ss1, si0, si1):
    cid = lax.axis_index("c")
    s = lax.axis_index("s")
    ZV = jnp.zeros((16,), _F32)
    A = (ks0, kd0, ex0, kc0, rows0, sg0, ss0, si0)
    B = (ks1, kd1, ex1, kc1, rows1, sg1, ss1, si1)

    def work(hx_h, agg_h):
        base0 = s * ET

        def fire_idx(c, P):
            ksb, kdb, exb, _, _, _, _, sip = P
            off = base0 + c * EC
            pltpu.async_copy(ks_h.at[pl.ds(off, EC)], ksb, sip)
            pltpu.async_copy(kd_h.at[pl.ds(off, EC)], kdb, sip)
            pltpu.async_copy(ex_h.at[pl.ds(off, EC)], exb, sip)

        def wait_idx(P):
            ksb, kdb, exb, _, _, _, _, sip = P
            pltpu.make_async_copy(ks_h.at[pl.ds(base0, EC)], ksb, sip).wait()
            pltpu.make_async_copy(kd_h.at[pl.ds(base0, EC)], kdb, sip).wait()
            pltpu.make_async_copy(ex_h.at[pl.ds(base0, EC)], exb, sip).wait()

        def fire_gather(P):
            ksb, _, _, _, rw, sgp, _, _ = P
            pltpu.async_copy(hx_h.at[ksb], rw, sgp)

        def wait_gather(P):
            ksb, _, _, _, rw, sgp, _, _ = P
            pltpu.make_async_copy(hx_h.at[ksb], rw, sgp).wait()

        def fire_scatter(P):
            _, _, _, kcp, rw, _, ssp, _ = P
            pltpu.async_copy(rw, acc.at[kcp], ssp, add=True)

        def wait_scatter(P):
            _, _, _, kcp, rw, _, ssp, _ = P
            pltpu.make_async_copy(rw, acc.at[kcp], ssp).wait()

        def scale(P):
            _, kdb, exb, kcp, rw, _, _, _ = P

            def rowf(r, _):
                wv = plsc.load_gather(exb, [jnp.full((16,), r, _I32)])
                for k in range(8):
                    idx = pl.ds(k * 16, 16)
                    rw[r, idx] = rw[r, idx] * wv
                return 0
            lax.fori_loop(0, EC, rowf, 0)
            for j in range(EC // 16):
                idx = pl.ds(j * 16, 16)
                kcp[idx] = kdb[idx]

        def zr(r, _):
            for k in range(8):
                rows0[r, pl.ds(k * 16, 16)] = ZV
            return 0
        lax.fori_loop(0, EC, zr, 0)
        for k in range(SL // EC):
            pltpu.sync_copy(rows0, acc.at[pl.ds(s * SL + k * EC, EC)])
        plsc.subcore_barrier()

        fire_idx(0, A)
        fire_idx(1, B)
        wait_idx(A)
        fire_gather(A)

        def pair(w, _):
            c0 = w * 2

            wait_gather(A)

            @pl.when(w > 0)
            def _():
                wait_scatter(B)
            wait_idx(B)
            fire_gather(B)
            scale(A)
            fire_scatter(A)
            fire_idx(c0 + 2, A)

            wait_gather(B)
            wait_scatter(A)
            wait_idx(A)
            fire_gather(A)
            scale(B)
            fire_scatter(B)

            @pl.when(w < (NCH - 1) // 2 - 1)
            def _():
                fire_idx(c0 + 3, B)
            return 0
        lax.fori_loop(0, (NCH - 1) // 2, pair, 0)

        wait_gather(A)
        wait_scatter(B)
        scale(A)
        fire_scatter(A)
        wait_scatter(A)

        plsc.subcore_barrier()
        pltpu.sync_copy(acc.at[pl.ds(s * SL, SL)], agg_h.at[pl.ds(s * SL, SL)])

    @pl.when(cid == 0)
    def _():
        work(hxl_h, aggl_h)

    @pl.when(cid == 1)
    def _():
        work(hxr_h, aggr_h)


def _conv(hxl, hxr, ks, kd, ex):
    k2 = pl.kernel(
        _conv_body,
        out_type=[
            jax.ShapeDtypeStruct((NP, 128), _F32),
            jax.ShapeDtypeStruct((NP, 128), _F32),
        ],
        mesh=plsc.VectorSubcoreMesh(
            core_axis_name="c", subcore_axis_name="s",
            num_cores=2, num_subcores=NT),
        scratch_types=[
            pltpu.VMEM((EC,), _I32),
            pltpu.VMEM((EC,), _I32),
            pltpu.VMEM((EC,), _F32),
            pltpu.VMEM((EC,), _I32),
            pltpu.VMEM((EC,), _I32),
            pltpu.VMEM((EC,), _F32),
            pltpu.VMEM((EC,), _I32),
            pltpu.VMEM((EC,), _I32),
            pltpu.VMEM((EC, 128), _F32),
            pltpu.VMEM((EC, 128), _F32),
            pltpu.VMEM_SHARED((NP, 128), _F32),
            pltpu.SemaphoreType.DMA,
            pltpu.SemaphoreType.DMA,
            pltpu.SemaphoreType.DMA,
            pltpu.SemaphoreType.DMA,
            pltpu.SemaphoreType.DMA,
            pltpu.SemaphoreType.DMA,
        ],
        compiler_params=pltpu.CompilerParams(needs_layout_passes=False),
    )
    return k2(hxl, hxr, ks, kd, ex)


def kernel(x, edge_index, W_in, b_in, W1, b1, W2, b2, W3, b3,
           W_out, b_out, att_w, att_b, wp):
    xp = jnp.pad(x, ((0, NP - N), (0, 0)))
    src_r = edge_index[0].reshape(NT, NCH, EC)
    dst_r = edge_index[1].reshape(NT, NCH, EC)
    Ws = jnp.pad(jnp.concatenate([wp, att_w[0:256], att_w[256:512]], axis=1),
                 ((0, 0), (0, 125)))
    apv = jnp.full((16,), att_w[512, 0], _F32)

    def b8(b):
        return jnp.broadcast_to(b[None, :], (8, b.shape[0]))

    xhl, xhr, scal = _prelude(xp, W_in, b8(b_in), Ws)
    sp = scal[:, 0]
    si = scal[:, 1] + att_b[0]
    sj = scal[:, 2]
    delta = scal[:, 3]

    ks, kd, ex, inv, totf = _edge_prep(src_r, dst_r, delta, sp, si, sj, apv)
    ks = ks.reshape(E)
    kd = kd.reshape(E)
    ex = ex.reshape(E)

    inv2 = inv.reshape(NP, 1)
    ones = jnp.ones((NP, 1), _F32)
    h1l, h1r = _layer(xhl, xhr, ones, W1, b8(b1))
    a1l, a1r = _conv(h1l, h1r, ks, kd, ex)
    h2l, h2r = _layer(a1l, a1r, inv2, W2, b8(b2))
    a2l, a2r = _conv(h2l, h2r, ks, kd, ex)
    h3l, h3r = _layer(a2l, a2r, inv2, W3, b8(b3))
    a3l, a3r = _conv(h3l, h3r, ks, kd, ex)
    out = _final(a3l, a3r, inv2, W_out, b8(b_out), totf.reshape(NP, 1))
    return out[:N]


# speedup vs baseline: 22.8189x; 1.0678x over previous
import functools

import jax
import jax.numpy as jnp
from jax import lax
from jax.experimental import pallas as pl
from jax.experimental.pallas import tpu as pltpu
from jax.experimental.pallas import tpu_sc as plsc

N = 10000
E = 160000
NP = 10240
SL = 640
NT = 16
ET = E // NT
EC = 80
NCH = ET // EC
WAVE = 5
EC2 = 128
NC2 = ET // EC2
TAIL = ET - NC2 * EC2
TAU = 0.9

_F32 = jnp.float32
_I32 = jnp.int32


def _dot(a, b):
    return jax.lax.dot_general(
        a, b, (((1,), (0,)), ((), ())),
        precision=jax.lax.Precision.HIGHEST,
        preferred_element_type=_F32)


def _prelude_body(x_ref, w_ref, b_ref, ws_ref, xhl_ref, xhr_ref, scal_ref):
    i = pl.program_id(0)
    xh = jnp.maximum(_dot(x_ref[...], w_ref[...]) + b_ref[0:1, :], 0.0)
    xhl_ref[...] = xh[:, :128]
    xhr_ref[...] = xh[:, 128:]
    s = _dot(xh, ws_ref[...])
    d = jnp.sum(jnp.abs(xh), axis=1, keepdims=True)
    col = jax.lax.broadcasted_iota(_I32, (1024, 128), 1)
    row = i * 1024 + jax.lax.broadcasted_iota(_I32, (1024, 128), 0)
    s = s + jnp.where(col == 3, d, 0.0)
    scal_ref[...] = jnp.where(row < N, s, -1e9)


def _prelude(xp, W_in, b_in, Ws):
    return pl.pallas_call(
        _prelude_body,
        grid=(10,),
        in_specs=[
            pl.BlockSpec((1024, 256), lambda i: (i, 0)),
            pl.BlockSpec((256, 256), lambda i: (0, 0)),
            pl.BlockSpec((8, 256), lambda i: (0, 0)),
            pl.BlockSpec((256, 128), lambda i: (0, 0)),
        ],
        out_specs=[
            pl.BlockSpec((1024, 128), lambda i: (i, 0)),
            pl.BlockSpec((1024, 128), lambda i: (i, 0)),
            pl.BlockSpec((1024, 128), lambda i: (i, 0)),
        ],
        out_shape=[
            jax.ShapeDtypeStruct((NP, 128), _F32),
            jax.ShapeDtypeStruct((NP, 128), _F32),
            jax.ShapeDtypeStruct((NP, 128), _F32),
        ],
    )(xp, W_in, b_in, Ws)


def _layer_body(hl_ref, hr_ref, inv_ref, w_ref, b_ref, ol_ref, or_ref):
    h = jnp.concatenate([hl_ref[...], hr_ref[...]], axis=1)
    h = jnp.maximum(h * inv_ref[...], 0.0)
    hx = _dot(h, w_ref[...]) + b_ref[0:1, :]
    ol_ref[...] = hx[:, :128]
    or_ref[...] = hx[:, 128:]


def _layer(hl, hr, inv, W, b):
    return pl.pallas_call(
        _layer_body,
        grid=(10,),
        in_specs=[
            pl.BlockSpec((1024, 128), lambda i: (i, 0)),
            pl.BlockSpec((1024, 128), lambda i: (i, 0)),
            pl.BlockSpec((1024, 1), lambda i: (i, 0)),
            pl.BlockSpec((256, 256), lambda i: (0, 0)),
            pl.BlockSpec((8, 256), lambda i: (0, 0)),
        ],
        out_specs=[
            pl.BlockSpec((1024, 128), lambda i: (i, 0)),
            pl.BlockSpec((1024, 128), lambda i: (i, 0)),
        ],
        out_shape=[
            jax.ShapeDtypeStruct((NP, 128), _F32),
            jax.ShapeDtypeStruct((NP, 128), _F32),
        ],
    )(hl, hr, inv, W, b)


def _final_body(hl_ref, hr_ref, inv_ref, w_ref, b_ref, m_ref, o_ref):
    h = jnp.concatenate([hl_ref[...], hr_ref[...]], axis=1)
    h = jnp.maximum(h * inv_ref[...], 0.0)
    o_ref[...] = (_dot(h, w_ref[...]) + b_ref[0:1, :]) * m_ref[...]


def _final(hl, hr, inv, W, b, mask):
    return pl.pallas_call(
        _final_body,
        grid=(10,),
        in_specs=[
            pl.BlockSpec((1024, 128), lambda i: (i, 0)),
            pl.BlockSpec((1024, 128), lambda i: (i, 0)),
            pl.BlockSpec((1024, 1), lambda i: (i, 0)),
            pl.BlockSpec((256, 128), lambda i: (0, 0)),
            pl.BlockSpec((8, 128), lambda i: (0, 0)),
            pl.BlockSpec((1024, 1), lambda i: (i, 0)),
        ],
        out_specs=pl.BlockSpec((1024, 128), lambda i: (i, 0)),
        out_shape=jax.ShapeDtypeStruct((NP, 128), _F32),
    )(hl, hr, inv, W, b, mask)


def _k1_body(src_h, dst_h, delta_h, sp_h, si_h, sj_h, ap_h,
             ks_o, kd_o, ex_o, inv_o, tot_o,
             src2, dst2, nodef, sjf, toti, ranki,
             ksb, kdb, exb, sp_s, tmp, tmpi, tots, zbf, s16, flat256, apv,
             accA, shf, shi, sh16, sem):
    t = lax.axis_index("s")
    sl = pl.ds(t * SL, SL)
    ZV = jnp.zeros((16,), _F32)
    ZI = jnp.zeros((16,), _I32)

    def share_i(slice_ref, full_ref):
        pltpu.sync_copy(slice_ref, shi.at[sl])
        plsc.subcore_barrier()
        pltpu.sync_copy(shi, full_ref)
        plsc.subcore_barrier()

    def share_f(slice_ref, full_ref):
        pltpu.sync_copy(slice_ref, shf.at[sl])
        plsc.subcore_barrier()
        pltpu.sync_copy(shf, full_ref)
        plsc.subcore_barrier()

    def fire_adds(val2, idx2, acc):
        for w in range(NCH // WAVE):
            ds_ = [pltpu.async_copy(val2.at[w * WAVE + b],
                                    acc.at[idx2.at[w * WAVE + b]],
                                    sem, add=True)
                   for b in range(WAVE)]
            for dd in ds_:
                dd.wait()

    pltpu.sync_copy(src_h.at[t], src2)
    pltpu.sync_copy(dst_h.at[t], dst2)
    pltpu.sync_copy(delta_h, nodef)
    pltpu.sync_copy(sp_h.at[sl], sp_s)
    pltpu.sync_copy(sj_h, sjf)
    pltpu.sync_copy(ap_h, apv)

    def zf(j, _):
        tmp[pl.ds(j * 16, 16)] = ZV
        zbf[pl.ds(j * 16, 16)] = ZV
        return 0
    lax.fori_loop(0, SL // 16, zf, 0)
    pltpu.sync_copy(tmp, accA.at[sl])
    plsc.subcore_barrier()

    def av(c, _):
        def iv(j, _):
            idx = pl.ds(j * 16, 16)
            exb[c, idx] = plsc.load_gather(nodef, [src2[c, idx]])
            return 0
        lax.fori_loop(0, EC // 16, iv, 0)
        return 0
    lax.fori_loop(0, NCH, av, 0)
    fire_adds(exb, dst2, accA)
    plsc.subcore_barrier()
    pltpu.sync_copy(accA.at[sl], tmp)
    pltpu.sync_copy(zbf, accA.at[sl])

    def pv(j, anyv):
        idx = pl.ds(j * 16, 16)
        x = sp_s[idx] + tmp[idx]
        p = 1.0 / (1.0 + jnp.exp(-x))
        sp_s[idx] = p
        selv = (p >= TAU).astype(_I32)
        tots[idx] = selv
        tmpi[idx] = selv
        return jnp.maximum(anyv, selv)
    anyv = lax.fori_loop(0, SL // 16, pv, ZI)
    s16[pl.ds(0, 16)] = anyv
    pltpu.sync_copy(s16, sh16.at[pl.ds(t * 16, 16)])
    share_i(tmpi, ranki)
    pltpu.sync_copy(sh16, flat256)

    def rmax(k, a):
        return jnp.maximum(a, flat256[pl.ds(k * 16, 16)])
    any_s = jnp.max(lax.fori_loop(0, 16, rmax, ZI))

    for _hop in range(2):
        def bv(c, _):
            def ivb(j, _):
                idx = pl.ds(j * 16, 16)
                exb[c, idx] = plsc.load_gather(
                    ranki, [dst2[c, idx]]).astype(_F32)
                return 0
            lax.fori_loop(0, EC // 16, ivb, 0)
            return 0
        lax.fori_loop(0, NCH, bv, 0)
        fire_adds(exb, src2, accA)
        plsc.subcore_barrier()
        pltpu.sync_copy(accA.at[sl], tmp)
        pltpu.sync_copy(zbf, accA.at[sl])

        def fv(j, _):
            idx = pl.ds(j * 16, 16)
            fn = (tmp[idx] > 0.0).astype(_I32)
            tots[idx] = tots[idx] | fn
            tmpi[idx] = fn
            return 0
        lax.fori_loop(0, SL // 16, fv, 0)
        share_i(tmpi, ranki)

    def tv(j, cnt):
        idx = pl.ds(j * 16, 16)
        v = jnp.where(any_s > 0, tots[idx], 1)
        tots[idx] = v
        return cnt + jnp.sum(v)
    cnt = lax.fori_loop(0, SL // 16, tv, jnp.int32(0))
    s16[pl.ds(0, 16)] = jnp.full((16,), cnt, _I32)
    pltpu.sync_copy(s16, sh16.at[pl.ds(t * 16, 16)])
    share_i(tots, toti)
    pltpu.sync_copy(sh16, flat256)
    lanes = lax.iota(_I32, 16)
    cnts = plsc.load_gather(flat256, [lanes * 16])
    off = jnp.sum(jnp.where(lanes < t, cnts, 0))

    def rv(j, carry):
        idx = pl.ds(j * 16, 16)
        v = tots[idx]
        tmpi[idx] = plsc.cumsum(v) + (off + carry - 1)
        return carry + jnp.sum(v)
    lax.fori_loop(0, SL // 16, rv, jnp.int32(0))
    share_i(tmpi, ranki)

    share_f(sp_s, nodef)
    apl = apv[pl.ds(0, 16)]

    def tj(j, _):
        idx = pl.ds(j * 16, 16)
        sjf[idx] = sjf[idx] + apl * nodef[idx]
        return 0
    lax.fori_loop(0, NP // 16, tj, 0)
    pltpu.sync_copy(si_h, nodef)

    def dvl(c, _):
        def ivd(j, _):
            idx = pl.ds(j * 16, 16)
            sv = src2[c, idx]
            dv = dst2[c, idx]
            ts = plsc.load_gather(toti, [sv])
            td = plsc.load_gather(toti, [dv])
            em = ts & td
            rs = plsc.load_gather(ranki, [sv])
            rd = plsc.load_gather(ranki, [dv])
            emb = em > 0
            ks0 = jnp.where(emb, rs, 0)
            kd0 = jnp.where(emb, rd, 0)
            tks = plsc.load_gather(toti, [ks0])
            tkd = plsc.load_gather(toti, [kd0])
            vab = (em & tks & tkd) > 0
            ksv = jnp.where(vab, ks0, 0)
            kdv = jnp.where(vab, kd0, 0)
            e = plsc.load_gather(nodef, [kdv]) + plsc.load_gather(sjf, [ksv])
            e = jnp.maximum(e, 0.2 * e)
            exv = jnp.where(vab, jnp.exp(e), 0.0)
            ksb[c, idx] = ksv
            kdb[c, idx] = kdv
            exb[c, idx] = exv
            return 0
        lax.fori_loop(0, EC // 16, ivd, 0)
        return 0
    lax.fori_loop(0, NCH, dvl, 0)
    fire_adds(exb, kdb, accA)
    plsc.subcore_barrier()

    pltpu.sync_copy(ksb, ks_o.at[t])
    pltpu.sync_copy(kdb, kd_o.at[t])
    pltpu.sync_copy(exb, ex_o.at[t])
    pltpu.sync_copy(accA.at[sl], tmp)

    def iv2(j, _):
        idx = pl.ds(j * 16, 16)
        tmp[idx] = 1.0 / (tmp[idx] + 1e-16)
        return 0
    lax.fori_loop(0, SL // 16, iv2, 0)
    pltpu.sync_copy(tmp, inv_o.at[sl])

    def tf2(j, _):
        idx = pl.ds(j * 16, 16)
        tmp[idx] = tots[idx].astype(_F32)
        return 0
    lax.fori_loop(0, SL // 16, tf2, 0)
    pltpu.sync_copy(tmp, tot_o.at[sl])


def _edge_prep(src_r, dst_r, delta, sp, si, sj, apv):
    k1 = pl.kernel(
        _k1_body,
        out_type=[
            jax.ShapeDtypeStruct((NT, NCH, EC), _I32),
            jax.ShapeDtypeStruct((NT, NCH, EC), _I32),
            jax.ShapeDtypeStruct((NT, NCH, EC), _F32),
            jax.ShapeDtypeStruct((NP,), _F32),
            jax.ShapeDtypeStruct((NP,), _F32),
        ],
        mesh=plsc.VectorSubcoreMesh(
            core_axis_name="c", subcore_axis_name="s",
            num_cores=1, num_subcores=NT),
        scratch_types=[
            pltpu.VMEM((NCH, EC), _I32),
            pltpu.VMEM((NCH, EC), _I32),
            pltpu.VMEM((NP,), _F32),
            pltpu.VMEM((NP,), _F32),
            pltpu.VMEM((NP,), _I32),
            pltpu.VMEM((NP,), _I32),
            pltpu.VMEM((NCH, EC), _I32),
            pltpu.VMEM((NCH, EC), _I32),
            pltpu.VMEM((NCH, EC), _F32),
            pltpu.VMEM((SL,), _F32),
            pltpu.VMEM((SL,), _F32),
            pltpu.VMEM((SL,), _I32),
            pltpu.VMEM((SL,), _I32),
            pltpu.VMEM((SL,), _F32),
            pltpu.VMEM((16,), _I32),
            pltpu.VMEM((256,), _I32),
            pltpu.VMEM((16,), _F32),
            pltpu.VMEM_SHARED((NP,), _F32),
            pltpu.VMEM_SHARED((NP,), _F32),
            pltpu.VMEM_SHARED((NP,), _I32),
            pltpu.VMEM_SHARED((256,), _I32),
            pltpu.SemaphoreType.DMA,
        ],
        compiler_params=pltpu.CompilerParams(needs_layout_passes=False),
    )
    return k1(src_r, dst_r, delta, sp, si, sj, apv)


def _conv_body(hxl_h, hxr_h, ks_h, kd_h, ex_h, aggl_h, aggr_h,
               ks0, kd0, ex0, ks1, kd1, ex1, kc0, kc1, rows0, rows1,
               kst, kdt, ext, acc, sg0, sg1, ss0, ss1, si0, si1):
    cid = lax.axis_index("c")
    s = lax.axis_index("s")
    ZV = jnp.zeros((16,), _F32)
    A = (ks0, kd0, ex0, kc0, rows0, sg0, ss0, si0)
    B = (ks1, kd1, ex1, kc1, rows1, sg1, ss1, si1)

    def work(hx_h, agg_h):
        base0 = s * ET

        def fire_idx(c, P):
            ksb, kdb, exb, _, _, _, _, sip = P
            off = base0 + c * EC2
            pltpu.async_copy(ks_h.at[pl.ds(off, EC2)], ksb, sip)
            pltpu.async_copy(kd_h.at[pl.ds(off, EC2)], kdb, sip)
            pltpu.async_copy(ex_h.at[pl.ds(off, EC2)], exb, sip)

        def wait_idx(P):
            ksb, kdb, exb, _, _, _, _, sip = P
            pltpu.make_async_copy(ks_h.at[pl.ds(base0, EC2)], ksb, sip).wait()
            pltpu.make_async_copy(kd_h.at[pl.ds(base0, EC2)], kdb, sip).wait()
            pltpu.make_async_copy(ex_h.at[pl.ds(base0, EC2)], exb, sip).wait()

        def fire_gather(P):
            ksb, _, _, _, rw, sgp, _, _ = P
            pltpu.async_copy(hx_h.at[ksb], rw, sgp)

        def wait_gather(P):
            ksb, _, _, _, rw, sgp, _, _ = P
            pltpu.make_async_copy(hx_h.at[ksb], rw, sgp).wait()

        def fire_scatter(P):
            _, _, _, kcp, rw, _, ssp, _ = P
            pltpu.async_copy(rw, acc.at[kcp], ssp, add=True)

        def wait_scatter(P):
            _, _, _, kcp, rw, _, ssp, _ = P
            pltpu.make_async_copy(rw, acc.at[kcp], ssp).wait()

        def scale(P):
            _, kdb, exb, kcp, rw, _, _, _ = P

            def rowf(rf, _):
                for rr in range(2):
                    r = rf * 2 + rr
                    wv = plsc.load_gather(exb, [jnp.full((16,), r, _I32)])
                    for k in range(8):
                        idx = pl.ds(k * 16, 16)
                        rw[r, idx] = rw[r, idx] * wv
                return 0
            lax.fori_loop(0, EC2 // 2, rowf, 0)
            for j in range(EC2 // 16):
                idx = pl.ds(j * 16, 16)
                kcp[idx] = kdb[idx]

        def zr(r, _):
            for k in range(8):
                rows0[r, pl.ds(k * 16, 16)] = ZV
            return 0
        lax.fori_loop(0, EC2, zr, 0)
        for k in range(SL // EC2):
            pltpu.sync_copy(rows0, acc.at[pl.ds(s * SL + k * EC2, EC2)])
        plsc.subcore_barrier()

        fire_idx(0, A)
        fire_idx(1, B)
        wait_idx(A)
        fire_gather(A)
        NPAIR = NC2 // 2

        def pair(w, _):
            c0 = w * 2

            wait_gather(A)

            @pl.when(w > 0)
            def _():
                wait_scatter(B)
            wait_idx(B)
            fire_gather(B)
            scale(A)
            fire_scatter(A)

            @pl.when(w < NPAIR - 1)
            def _():
                fire_idx(c0 + 2, A)

            wait_gather(B)
            wait_scatter(A)

            @pl.when(w < NPAIR - 1)
            def _():
                wait_idx(A)
                fire_gather(A)
            scale(B)
            fire_scatter(B)

            @pl.when(w < NPAIR - 1)
            def _():
                fire_idx(c0 + 3, B)
            return 0
        lax.fori_loop(0, NPAIR, pair, 0)
        wait_scatter(B)

        toff = base0 + NC2 * EC2
        pltpu.sync_copy(ks_h.at[pl.ds(toff, TAIL)], kst)
        pltpu.sync_copy(kd_h.at[pl.ds(toff, TAIL)], kdt)
        pltpu.sync_copy(ex_h.at[pl.ds(toff, TAIL)], ext)
        pltpu.async_copy(hx_h.at[kst], rows0.at[pl.ds(0, TAIL)], sg0).wait()

        def rowt(r, _):
            wv = plsc.load_gather(ext, [jnp.full((16,), r, _I32)])
            for k in range(8):
                idx = pl.ds(k * 16, 16)
                rows0[r, idx] = rows0[r, idx] * wv
            return 0
        lax.fori_loop(0, TAIL, rowt, 0)
        pltpu.async_copy(rows0.at[pl.ds(0, TAIL)], acc.at[kdt],
                         ss0, add=True).wait()

        plsc.subcore_barrier()
        pltpu.sync_copy(acc.at[pl.ds(s * SL, SL)], agg_h.at[pl.ds(s * SL, SL)])

    @pl.when(cid == 0)
    def _():
        work(hxl_h, aggl_h)

    @pl.when(cid == 1)
    def _():
        work(hxr_h, aggr_h)


def _conv(hxl, hxr, ks, kd, ex):
    k2 = pl.kernel(
        _conv_body,
        out_type=[
            jax.ShapeDtypeStruct((NP, 128), _F32),
            jax.ShapeDtypeStruct((NP, 128), _F32),
        ],
        mesh=plsc.VectorSubcoreMesh(
            core_axis_name="c", subcore_axis_name="s",
            num_cores=2, num_subcores=NT),
        scratch_types=[
            pltpu.VMEM((EC2,), _I32),
            pltpu.VMEM((EC2,), _I32),
            pltpu.VMEM((EC2,), _F32),
            pltpu.VMEM((EC2,), _I32),
            pltpu.VMEM((EC2,), _I32),
            pltpu.VMEM((EC2,), _F32),
            pltpu.VMEM((EC2,), _I32),
            pltpu.VMEM((EC2,), _I32),
            pltpu.VMEM((EC2, 128), _F32),
            pltpu.VMEM((EC2, 128), _F32),
            pltpu.VMEM((TAIL,), _I32),
            pltpu.VMEM((TAIL,), _I32),
            pltpu.VMEM((TAIL,), _F32),
            pltpu.VMEM_SHARED((NP, 128), _F32),
            pltpu.SemaphoreType.DMA,
            pltpu.SemaphoreType.DMA,
            pltpu.SemaphoreType.DMA,
            pltpu.SemaphoreType.DMA,
            pltpu.SemaphoreType.DMA,
            pltpu.SemaphoreType.DMA,
        ],
        compiler_params=pltpu.CompilerParams(needs_layout_passes=False),
    )
    return k2(hxl, hxr, ks, kd, ex)


def kernel(x, edge_index, W_in, b_in, W1, b1, W2, b2, W3, b3,
           W_out, b_out, att_w, att_b, wp):
    xp = jnp.pad(x, ((0, NP - N), (0, 0)))
    src_r = edge_index[0].reshape(NT, NCH, EC)
    dst_r = edge_index[1].reshape(NT, NCH, EC)
    Ws = jnp.pad(jnp.concatenate([wp, att_w[0:256], att_w[256:512]], axis=1),
                 ((0, 0), (0, 125)))
    apv = jnp.full((16,), att_w[512, 0], _F32)

    def b8(b):
        return jnp.broadcast_to(b[None, :], (8, b.shape[0]))

    xhl, xhr, scal = _prelude(xp, W_in, b8(b_in), Ws)
    sp = scal[:, 0]
    si = scal[:, 1] + att_b[0]
    sj = scal[:, 2]
    delta = scal[:, 3]

    ks, kd, ex, inv, totf = _edge_prep(src_r, dst_r, delta, sp, si, sj, apv)
    ks = ks.reshape(E)
    kd = kd.reshape(E)
    ex = ex.reshape(E)

    inv2 = inv.reshape(NP, 1)
    ones = jnp.ones((NP, 1), _F32)
    h1l, h1r = _layer(xhl, xhr, ones, W1, b8(b1))
    a1l, a1r = _conv(h1l, h1r, ks, kd, ex)
    h2l, h2r = _layer(a1l, a1r, inv2, W2, b8(b2))
    a2l, a2r = _conv(h2l, h2r, ks, kd, ex)
    h3l, h3r = _layer(a2l, a2r, inv2, W3, b8(b3))
    a3l, a3r = _conv(h3l, h3r, ks, kd, ex)
    out = _final(a3l, a3r, inv2, W_out, b8(b_out), totf.reshape(NP, 1))
    return out[:N]


# speedup vs baseline: 23.0387x; 1.0096x over previous
import functools

import jax
import jax.numpy as jnp
from jax import lax
from jax.experimental import pallas as pl
from jax.experimental.pallas import tpu as pltpu
from jax.experimental.pallas import tpu_sc as plsc

N = 10000
E = 160000
NP = 10240
SL = 640
NT = 16
ET = E // NT
EC = 80
NCH = ET // EC
WAVE = 5
EC2 = 128
NC2 = ET // EC2
TAIL = ET - NC2 * EC2
TAU = 0.9

_F32 = jnp.float32
_I32 = jnp.int32


def _dot(a, b):
    return jax.lax.dot_general(
        a, b, (((1,), (0,)), ((), ())),
        precision=jax.lax.Precision.HIGHEST,
        preferred_element_type=_F32)


def _prelude_body(x_ref, w_ref, b_ref, ws_ref, w1_ref, b1_ref,
                  h1l_ref, h1r_ref, scal_ref):
    i = pl.program_id(0)
    xh = jnp.maximum(_dot(x_ref[...], w_ref[...]) + b_ref[0:1, :], 0.0)
    hx1 = _dot(xh, w1_ref[...]) + b1_ref[0:1, :]
    h1l_ref[...] = hx1[:, :128]
    h1r_ref[...] = hx1[:, 128:]
    s = _dot(xh, ws_ref[...])
    d = jnp.sum(jnp.abs(xh), axis=1, keepdims=True)
    col = jax.lax.broadcasted_iota(_I32, (1024, 128), 1)
    row = i * 1024 + jax.lax.broadcasted_iota(_I32, (1024, 128), 0)
    s = s + jnp.where(col == 3, d, 0.0)
    scal_ref[...] = jnp.where(row < N, s, -1e9)


def _prelude(xp, W_in, b_in, Ws, W1, b1):
    return pl.pallas_call(
        _prelude_body,
        grid=(10,),
        in_specs=[
            pl.BlockSpec((1024, 256), lambda i: (i, 0)),
            pl.BlockSpec((256, 256), lambda i: (0, 0)),
            pl.BlockSpec((8, 256), lambda i: (0, 0)),
            pl.BlockSpec((256, 128), lambda i: (0, 0)),
            pl.BlockSpec((256, 256), lambda i: (0, 0)),
            pl.BlockSpec((8, 256), lambda i: (0, 0)),
        ],
        out_specs=[
            pl.BlockSpec((1024, 128), lambda i: (i, 0)),
            pl.BlockSpec((1024, 128), lambda i: (i, 0)),
            pl.BlockSpec((1024, 128), lambda i: (i, 0)),
        ],
        out_shape=[
            jax.ShapeDtypeStruct((NP, 128), _F32),
            jax.ShapeDtypeStruct((NP, 128), _F32),
            jax.ShapeDtypeStruct((NP, 128), _F32),
        ],
    )(xp, W_in, b_in, Ws, W1, b1)


def _layer_body(hl_ref, hr_ref, inv_ref, w_ref, b_ref, ol_ref, or_ref):
    h = jnp.concatenate([hl_ref[...], hr_ref[...]], axis=1)
    h = jnp.maximum(h * inv_ref[...], 0.0)
    hx = _dot(h, w_ref[...]) + b_ref[0:1, :]
    ol_ref[...] = hx[:, :128]
    or_ref[...] = hx[:, 128:]


def _layer(hl, hr, inv, W, b):
    return pl.pallas_call(
        _layer_body,
        grid=(10,),
        in_specs=[
            pl.BlockSpec((1024, 128), lambda i: (i, 0)),
            pl.BlockSpec((1024, 128), lambda i: (i, 0)),
            pl.BlockSpec((1024, 1), lambda i: (i, 0)),
            pl.BlockSpec((256, 256), lambda i: (0, 0)),
            pl.BlockSpec((8, 256), lambda i: (0, 0)),
        ],
        out_specs=[
            pl.BlockSpec((1024, 128), lambda i: (i, 0)),
            pl.BlockSpec((1024, 128), lambda i: (i, 0)),
        ],
        out_shape=[
            jax.ShapeDtypeStruct((NP, 128), _F32),
            jax.ShapeDtypeStruct((NP, 128), _F32),
        ],
    )(hl, hr, inv, W, b)


def _final_body(hl_ref, hr_ref, inv_ref, w_ref, b_ref, m_ref, o_ref):
    h = jnp.concatenate([hl_ref[...], hr_ref[...]], axis=1)
    h = jnp.maximum(h * inv_ref[...], 0.0)
    o_ref[...] = (_dot(h, w_ref[...]) + b_ref[0:1, :]) * m_ref[...]


def _final(hl, hr, inv, W, b, mask):
    return pl.pallas_call(
        _final_body,
        grid=(10,),
        in_specs=[
            pl.BlockSpec((1024, 128), lambda i: (i, 0)),
            pl.BlockSpec((1024, 128), lambda i: (i, 0)),
            pl.BlockSpec((1024, 1), lambda i: (i, 0)),
            pl.BlockSpec((256, 128), lambda i: (0, 0)),
            pl.BlockSpec((8, 128), lambda i: (0, 0)),
            pl.BlockSpec((1024, 1), lambda i: (i, 0)),
        ],
        out_specs=pl.BlockSpec((1024, 128), lambda i: (i, 0)),
        out_shape=jax.ShapeDtypeStruct((NP, 128), _F32),
    )(hl, hr, inv, W, b, mask)


def _k1_body(src_h, dst_h, delta_h, sp_h, si_h, sj_h, ap_h,
             ks_o, kd_o, ex_o, inv_o, tot_o,
             src2, dst2, nodef, sjf, toti, ranki,
             ksb, kdb, exb, sp_s, tmp, tmpi, tots, zbf, s16, flat256, apv,
             accA, shf, shi, sh16, sem):
    t = lax.axis_index("s")
    sl = pl.ds(t * SL, SL)
    ZV = jnp.zeros((16,), _F32)
    ZI = jnp.zeros((16,), _I32)

    def share_i(slice_ref, full_ref):
        pltpu.sync_copy(slice_ref, shi.at[sl])
        plsc.subcore_barrier()
        pltpu.sync_copy(shi, full_ref)
        plsc.subcore_barrier()

    def share_f(slice_ref, full_ref):
        pltpu.sync_copy(slice_ref, shf.at[sl])
        plsc.subcore_barrier()
        pltpu.sync_copy(shf, full_ref)
        plsc.subcore_barrier()

    def fire_adds(val2, idx2, acc):
        for w in range(NCH // WAVE):
            ds_ = [pltpu.async_copy(val2.at[w * WAVE + b],
                                    acc.at[idx2.at[w * WAVE + b]],
                                    sem, add=True)
                   for b in range(WAVE)]
            for dd in ds_:
                dd.wait()

    pltpu.sync_copy(src_h.at[t], src2)
    pltpu.sync_copy(dst_h.at[t], dst2)
    pltpu.sync_copy(delta_h, nodef)
    pltpu.sync_copy(sp_h.at[sl], sp_s)
    pltpu.sync_copy(sj_h, sjf)
    pltpu.sync_copy(ap_h, apv)

    def zf(j, _):
        tmp[pl.ds(j * 16, 16)] = ZV
        zbf[pl.ds(j * 16, 16)] = ZV
        return 0
    lax.fori_loop(0, SL // 16, zf, 0)
    pltpu.sync_copy(tmp, accA.at[sl])
    plsc.subcore_barrier()

    def av(c, _):
        def iv(j, _):
            idx = pl.ds(j * 16, 16)
            exb[c, idx] = plsc.load_gather(nodef, [src2[c, idx]])
            return 0
        lax.fori_loop(0, EC // 16, iv, 0)
        return 0
    lax.fori_loop(0, NCH, av, 0)
    fire_adds(exb, dst2, accA)
    plsc.subcore_barrier()
    pltpu.sync_copy(accA.at[sl], tmp)
    pltpu.sync_copy(zbf, accA.at[sl])

    def pv(j, anyv):
        idx = pl.ds(j * 16, 16)
        x = sp_s[idx] + tmp[idx]
        p = 1.0 / (1.0 + jnp.exp(-x))
        sp_s[idx] = p
        selv = (p >= TAU).astype(_I32)
        tots[idx] = selv
        tmpi[idx] = selv
        return jnp.maximum(anyv, selv)
    anyv = lax.fori_loop(0, SL // 16, pv, ZI)
    s16[pl.ds(0, 16)] = anyv
    pltpu.sync_copy(s16, sh16.at[pl.ds(t * 16, 16)])
    share_i(tmpi, ranki)
    pltpu.sync_copy(sh16, flat256)

    def rmax(k, a):
        return jnp.maximum(a, flat256[pl.ds(k * 16, 16)])
    any_s = jnp.max(lax.fori_loop(0, 16, rmax, ZI))

    for _hop in range(2):
        def bv(c, _):
            def ivb(j, _):
                idx = pl.ds(j * 16, 16)
                exb[c, idx] = plsc.load_gather(
                    ranki, [dst2[c, idx]]).astype(_F32)
                return 0
            lax.fori_loop(0, EC // 16, ivb, 0)
            return 0
        lax.fori_loop(0, NCH, bv, 0)
        fire_adds(exb, src2, accA)
        plsc.subcore_barrier()
        pltpu.sync_copy(accA.at[sl], tmp)
        pltpu.sync_copy(zbf, accA.at[sl])

        def fv(j, _):
            idx = pl.ds(j * 16, 16)
            fn = (tmp[idx] > 0.0).astype(_I32)
            tots[idx] = tots[idx] | fn
            tmpi[idx] = fn
            return 0
        lax.fori_loop(0, SL // 16, fv, 0)
        share_i(tmpi, ranki)

    def tv(j, cnt):
        idx = pl.ds(j * 16, 16)
        v = jnp.where(any_s > 0, tots[idx], 1)
        tots[idx] = v
        return cnt + jnp.sum(v)
    cnt = lax.fori_loop(0, SL // 16, tv, jnp.int32(0))
    s16[pl.ds(0, 16)] = jnp.full((16,), cnt, _I32)
    pltpu.sync_copy(s16, sh16.at[pl.ds(t * 16, 16)])
    share_i(tots, toti)
    pltpu.sync_copy(sh16, flat256)
    lanes = lax.iota(_I32, 16)
    cnts = plsc.load_gather(flat256, [lanes * 16])
    off = jnp.sum(jnp.where(lanes < t, cnts, 0))

    def rv(j, carry):
        idx = pl.ds(j * 16, 16)
        v = tots[idx]
        tmpi[idx] = plsc.cumsum(v) + (off + carry - 1)
        return carry + jnp.sum(v)
    lax.fori_loop(0, SL // 16, rv, jnp.int32(0))
    share_i(tmpi, ranki)

    share_f(sp_s, nodef)
    apl = apv[pl.ds(0, 16)]

    def tj(j, _):
        idx = pl.ds(j * 16, 16)
        sjf[idx] = sjf[idx] + apl * nodef[idx]
        return 0
    lax.fori_loop(0, NP // 16, tj, 0)
    pltpu.sync_copy(si_h, nodef)

    def dvl(c, _):
        def ivd(j, _):
            idx = pl.ds(j * 16, 16)
            sv = src2[c, idx]
            dv = dst2[c, idx]
            ts = plsc.load_gather(toti, [sv])
            td = plsc.load_gather(toti, [dv])
            em = ts & td
            rs = plsc.load_gather(ranki, [sv])
            rd = plsc.load_gather(ranki, [dv])
            emb = em > 0
            ks0 = jnp.where(emb, rs, 0)
            kd0 = jnp.where(emb, rd, 0)
            tks = plsc.load_gather(toti, [ks0])
            tkd = plsc.load_gather(toti, [kd0])
            vab = (em & tks & tkd) > 0
            ksv = jnp.where(vab, ks0, 0)
            kdv = jnp.where(vab, kd0, 0)
            e = plsc.load_gather(nodef, [kdv]) + plsc.load_gather(sjf, [ksv])
            e = jnp.maximum(e, 0.2 * e)
            exv = jnp.where(vab, jnp.exp(e), 0.0)
            ksb[c, idx] = ksv
            kdb[c, idx] = kdv
            exb[c, idx] = exv
            return 0
        lax.fori_loop(0, EC // 16, ivd, 0)
        return 0
    lax.fori_loop(0, NCH, dvl, 0)
    fire_adds(exb, kdb, accA)
    plsc.subcore_barrier()

    pltpu.sync_copy(ksb, ks_o.at[t])
    pltpu.sync_copy(kdb, kd_o.at[t])
    pltpu.sync_copy(exb, ex_o.at[t])
    pltpu.sync_copy(accA.at[sl], tmp)

    def iv2(j, _):
        idx = pl.ds(j * 16, 16)
        tmp[idx] = 1.0 / (tmp[idx] + 1e-16)
        return 0
    lax.fori_loop(0, SL // 16, iv2, 0)
    pltpu.sync_copy(tmp, inv_o.at[sl])

    def tf2(j, _):
        idx = pl.ds(j * 16, 16)
        tmp[idx] = tots[idx].astype(_F32)
        return 0
    lax.fori_loop(0, SL // 16, tf2, 0)
    pltpu.sync_copy(tmp, tot_o.at[sl])


def _edge_prep(src_r, dst_r, delta, sp, si, sj, apv):
    k1 = pl.kernel(
        _k1_body,
        out_type=[
            jax.ShapeDtypeStruct((NT, NCH, EC), _I32),
            jax.ShapeDtypeStruct((NT, NCH, EC), _I32),
            jax.ShapeDtypeStruct((NT, NCH, EC), _F32),
            jax.ShapeDtypeStruct((NP,), _F32),
            jax.ShapeDtypeStruct((NP,), _F32),
        ],
        mesh=plsc.VectorSubcoreMesh(
            core_axis_name="c", subcore_axis_name="s",
            num_cores=1, num_subcores=NT),
        scratch_types=[
            pltpu.VMEM((NCH, EC), _I32),
            pltpu.VMEM((NCH, EC), _I32),
            pltpu.VMEM((NP,), _F32),
            pltpu.VMEM((NP,), _F32),
            pltpu.VMEM((NP,), _I32),
            pltpu.VMEM((NP,), _I32),
            pltpu.VMEM((NCH, EC), _I32),
            pltpu.VMEM((NCH, EC), _I32),
            pltpu.VMEM((NCH, EC), _F32),
            pltpu.VMEM((SL,), _F32),
            pltpu.VMEM((SL,), _F32),
            pltpu.VMEM((SL,), _I32),
            pltpu.VMEM((SL,), _I32),
            pltpu.VMEM((SL,), _F32),
            pltpu.VMEM((16,), _I32),
            pltpu.VMEM((256,), _I32),
            pltpu.VMEM((16,), _F32),
            pltpu.VMEM_SHARED((NP,), _F32),
            pltpu.VMEM_SHARED((NP,), _F32),
            pltpu.VMEM_SHARED((NP,), _I32),
            pltpu.VMEM_SHARED((256,), _I32),
            pltpu.SemaphoreType.DMA,
        ],
        compiler_params=pltpu.CompilerParams(needs_layout_passes=False),
    )
    return k1(src_r, dst_r, delta, sp, si, sj, apv)


def _conv_body(hxl_h, hxr_h, ks_h, kd_h, ex_h, aggl_h, aggr_h,
               ks0, kd0, ex0, ks1, kd1, ex1, kc0, kc1, rows0, rows1,
               kst, kdt, ext, acc, sg0, sg1, ss0, ss1, si0, si1):
    cid = lax.axis_index("c")
    s = lax.axis_index("s")
    ZV = jnp.zeros((16,), _F32)
    A = (ks0, kd0, ex0, kc0, rows0, sg0, ss0, si0)
    B = (ks1, kd1, ex1, kc1, rows1, sg1, ss1, si1)

    def work(hx_h, agg_h):
        base0 = s * ET

        def fire_idx(c, P):
            ksb, kdb, exb, _, _, _, _, sip = P
            off = base0 + c * EC2
            pltpu.async_copy(ks_h.at[pl.ds(off, EC2)], ksb, sip)
            pltpu.async_copy(kd_h.at[pl.ds(off, EC2)], kdb, sip)
            pltpu.async_copy(ex_h.at[pl.ds(off, EC2)], exb, sip)

        def wait_idx(P):
            ksb, kdb, exb, _, _, _, _, sip = P
            pltpu.make_async_copy(ks_h.at[pl.ds(base0, EC2)], ksb, sip).wait()
            pltpu.make_async_copy(kd_h.at[pl.ds(base0, EC2)], kdb, sip).wait()
            pltpu.make_async_copy(ex_h.at[pl.ds(base0, EC2)], exb, sip).wait()

        def fire_gather(P):
            ksb, _, _, _, rw, sgp, _, _ = P
            pltpu.async_copy(hx_h.at[ksb], rw, sgp)

        def wait_gather(P):
            ksb, _, _, _, rw, sgp, _, _ = P
            pltpu.make_async_copy(hx_h.at[ksb], rw, sgp).wait()

        def fire_scatter(P):
            _, _, _, kcp, rw, _, ssp, _ = P
            pltpu.async_copy(rw, acc.at[kcp], ssp, add=True)

        def wait_scatter(P):
            _, _, _, kcp, rw, _, ssp, _ = P
            pltpu.make_async_copy(rw, acc.at[kcp], ssp).wait()

        def scale(P):
            _, kdb, exb, kcp, rw, _, _, _ = P

            def rowf(rf, _):
                for rr in range(2):
                    r = rf * 2 + rr
                    wv = plsc.load_gather(exb, [jnp.full((16,), r, _I32)])
                    for k in range(8):
                        idx = pl.ds(k * 16, 16)
                        rw[r, idx] = rw[r, idx] * wv
                return 0
            lax.fori_loop(0, EC2 // 2, rowf, 0)
            for j in range(EC2 // 16):
                idx = pl.ds(j * 16, 16)
                kcp[idx] = kdb[idx]

        def zr(r, _):
            for k in range(8):
                rows0[r, pl.ds(k * 16, 16)] = ZV
            return 0
        lax.fori_loop(0, EC2, zr, 0)
        for k in range(SL // EC2):
            pltpu.sync_copy(rows0, acc.at[pl.ds(s * SL + k * EC2, EC2)])
        plsc.subcore_barrier()

        fire_idx(0, A)
        fire_idx(1, B)
        wait_idx(A)
        fire_gather(A)
        NPAIR = NC2 // 2

        def pair(w, _):
            c0 = w * 2

            wait_gather(A)

            @pl.when(w > 0)
            def _():
                wait_scatter(B)
            wait_idx(B)
            fire_gather(B)
            scale(A)
            fire_scatter(A)

            @pl.when(w < NPAIR - 1)
            def _():
                fire_idx(c0 + 2, A)

            wait_gather(B)
            wait_scatter(A)

            @pl.when(w < NPAIR - 1)
            def _():
                wait_idx(A)
                fire_gather(A)
            scale(B)
            fire_scatter(B)

            @pl.when(w < NPAIR - 1)
            def _():
                fire_idx(c0 + 3, B)
            return 0
        lax.fori_loop(0, NPAIR, pair, 0)
        wait_scatter(B)

        toff = base0 + NC2 * EC2
        pltpu.sync_copy(ks_h.at[pl.ds(toff, TAIL)], kst)
        pltpu.sync_copy(kd_h.at[pl.ds(toff, TAIL)], kdt)
        pltpu.sync_copy(ex_h.at[pl.ds(toff, TAIL)], ext)
        pltpu.async_copy(hx_h.at[kst], rows0.at[pl.ds(0, TAIL)], sg0).wait()

        def rowt(r, _):
            wv = plsc.load_gather(ext, [jnp.full((16,), r, _I32)])
            for k in range(8):
                idx = pl.ds(k * 16, 16)
                rows0[r, idx] = rows0[r, idx] * wv
            return 0
        lax.fori_loop(0, TAIL, rowt, 0)
        pltpu.async_copy(rows0.at[pl.ds(0, TAIL)], acc.at[kdt],
                         ss0, add=True).wait()

        plsc.subcore_barrier()
        pltpu.sync_copy(acc.at[pl.ds(s * SL, SL)], agg_h.at[pl.ds(s * SL, SL)])

    @pl.when(cid == 0)
    def _():
        work(hxl_h, aggl_h)

    @pl.when(cid == 1)
    def _():
        work(hxr_h, aggr_h)


def _conv(hxl, hxr, ks, kd, ex):
    k2 = pl.kernel(
        _conv_body,
        out_type=[
            jax.ShapeDtypeStruct((NP, 128), _F32),
            jax.ShapeDtypeStruct((NP, 128), _F32),
        ],
        mesh=plsc.VectorSubcoreMesh(
            core_axis_name="c", subcore_axis_name="s",
            num_cores=2, num_subcores=NT),
        scratch_types=[
            pltpu.VMEM((EC2,), _I32),
            pltpu.VMEM((EC2,), _I32),
            pltpu.VMEM((EC2,), _F32),
            pltpu.VMEM((EC2,), _I32),
            pltpu.VMEM((EC2,), _I32),
            pltpu.VMEM((EC2,), _F32),
            pltpu.VMEM((EC2,), _I32),
            pltpu.VMEM((EC2,), _I32),
            pltpu.VMEM((EC2, 128), _F32),
            pltpu.VMEM((EC2, 128), _F32),
            pltpu.VMEM((TAIL,), _I32),
            pltpu.VMEM((TAIL,), _I32),
            pltpu.VMEM((TAIL,), _F32),
            pltpu.VMEM_SHARED((NP, 128), _F32),
            pltpu.SemaphoreType.DMA,
            pltpu.SemaphoreType.DMA,
            pltpu.SemaphoreType.DMA,
            pltpu.SemaphoreType.DMA,
            pltpu.SemaphoreType.DMA,
            pltpu.SemaphoreType.DMA,
        ],
        compiler_params=pltpu.CompilerParams(needs_layout_passes=False),
    )
    return k2(hxl, hxr, ks, kd, ex)


def kernel(x, edge_index, W_in, b_in, W1, b1, W2, b2, W3, b3,
           W_out, b_out, att_w, att_b, wp):
    xp = jnp.pad(x, ((0, NP - N), (0, 0)))
    src_r = edge_index[0].reshape(NT, NCH, EC)
    dst_r = edge_index[1].reshape(NT, NCH, EC)
    Ws = jnp.pad(jnp.concatenate([wp, att_w[0:256], att_w[256:512]], axis=1),
                 ((0, 0), (0, 125)))
    apv = jnp.full((16,), att_w[512, 0], _F32)

    def b8(b):
        return jnp.broadcast_to(b[None, :], (8, b.shape[0]))

    h1l, h1r, scal = _prelude(xp, W_in, b8(b_in), Ws, W1, b8(b1))
    sp = scal[:, 0]
    si = scal[:, 1] + att_b[0]
    sj = scal[:, 2]
    delta = scal[:, 3]

    ks, kd, ex, inv, totf = _edge_prep(src_r, dst_r, delta, sp, si, sj, apv)
    ks = ks.reshape(E)
    kd = kd.reshape(E)
    ex = ex.reshape(E)

    inv2 = inv.reshape(NP, 1)
    a1l, a1r = _conv(h1l, h1r, ks, kd, ex)
    h2l, h2r = _layer(a1l, a1r, inv2, W2, b8(b2))
    a2l, a2r = _conv(h2l, h2r, ks, kd, ex)
    h3l, h3r = _layer(a2l, a2r, inv2, W3, b8(b3))
    a3l, a3r = _conv(h3l, h3r, ks, kd, ex)
    out = _final(a3l, a3r, inv2, W_out, b8(b_out), totf.reshape(NP, 1))
    return out[:N]


# speedup vs baseline: 23.5224x; 1.0210x over previous
import functools

import jax
import jax.numpy as jnp
from jax import lax
from jax.experimental import pallas as pl
from jax.experimental.pallas import tpu as pltpu
from jax.experimental.pallas import tpu_sc as plsc

N = 10000
E = 160000
NP = 10240
SL = 640
NT = 16
ET = E // NT
EC = 80
NCH = ET // EC
WAVE = 25
EC2 = 128
NC2 = ET // EC2
TAIL = ET - NC2 * EC2
TAU = 0.9

_F32 = jnp.float32
_I32 = jnp.int32


def _dot(a, b):
    return jax.lax.dot_general(
        a, b, (((1,), (0,)), ((), ())),
        precision=jax.lax.Precision.HIGHEST,
        preferred_element_type=_F32)


def _prelude_body(x_ref, w_ref, b_ref, ws_ref, w1_ref, b1_ref,
                  h1l_ref, h1r_ref, scal_ref):
    i = pl.program_id(0)
    xh = jnp.maximum(_dot(x_ref[...], w_ref[...]) + b_ref[0:1, :], 0.0)
    hx1 = _dot(xh, w1_ref[...]) + b1_ref[0:1, :]
    h1l_ref[...] = hx1[:, :128]
    h1r_ref[...] = hx1[:, 128:]
    s = _dot(xh, ws_ref[...])
    d = jnp.sum(jnp.abs(xh), axis=1, keepdims=True)
    col = jax.lax.broadcasted_iota(_I32, (1024, 128), 1)
    row = i * 1024 + jax.lax.broadcasted_iota(_I32, (1024, 128), 0)
    s = s + jnp.where(col == 3, d, 0.0)
    scal_ref[...] = jnp.where(row < N, s, -1e9)


def _prelude(xp, W_in, b_in, Ws, W1, b1):
    return pl.pallas_call(
        _prelude_body,
        grid=(10,),
        in_specs=[
            pl.BlockSpec((1024, 256), lambda i: (i, 0)),
            pl.BlockSpec((256, 256), lambda i: (0, 0)),
            pl.BlockSpec((8, 256), lambda i: (0, 0)),
            pl.BlockSpec((256, 128), lambda i: (0, 0)),
            pl.BlockSpec((256, 256), lambda i: (0, 0)),
            pl.BlockSpec((8, 256), lambda i: (0, 0)),
        ],
        out_specs=[
            pl.BlockSpec((1024, 128), lambda i: (i, 0)),
            pl.BlockSpec((1024, 128), lambda i: (i, 0)),
            pl.BlockSpec((1024, 128), lambda i: (i, 0)),
        ],
        out_shape=[
            jax.ShapeDtypeStruct((NP, 128), _F32),
            jax.ShapeDtypeStruct((NP, 128), _F32),
            jax.ShapeDtypeStruct((NP, 128), _F32),
        ],
    )(xp, W_in, b_in, Ws, W1, b1)


def _layer_body(hl_ref, hr_ref, inv_ref, w_ref, b_ref, ol_ref, or_ref):
    h = jnp.concatenate([hl_ref[...], hr_ref[...]], axis=1)
    h = jnp.maximum(h * inv_ref[...], 0.0)
    hx = _dot(h, w_ref[...]) + b_ref[0:1, :]
    ol_ref[...] = hx[:, :128]
    or_ref[...] = hx[:, 128:]


def _layer(hl, hr, inv, W, b):
    return pl.pallas_call(
        _layer_body,
        grid=(10,),
        in_specs=[
            pl.BlockSpec((1024, 128), lambda i: (i, 0)),
            pl.BlockSpec((1024, 128), lambda i: (i, 0)),
            pl.BlockSpec((1024, 1), lambda i: (i, 0)),
            pl.BlockSpec((256, 256), lambda i: (0, 0)),
            pl.BlockSpec((8, 256), lambda i: (0, 0)),
        ],
        out_specs=[
            pl.BlockSpec((1024, 128), lambda i: (i, 0)),
            pl.BlockSpec((1024, 128), lambda i: (i, 0)),
        ],
        out_shape=[
            jax.ShapeDtypeStruct((NP, 128), _F32),
            jax.ShapeDtypeStruct((NP, 128), _F32),
        ],
    )(hl, hr, inv, W, b)


def _final_body(hl_ref, hr_ref, inv_ref, w_ref, b_ref, m_ref, o_ref):
    h = jnp.concatenate([hl_ref[...], hr_ref[...]], axis=1)
    h = jnp.maximum(h * inv_ref[...], 0.0)
    o_ref[...] = (_dot(h, w_ref[...]) + b_ref[0:1, :]) * m_ref[...]


def _final(hl, hr, inv, W, b, mask):
    return pl.pallas_call(
        _final_body,
        grid=(10,),
        in_specs=[
            pl.BlockSpec((1024, 128), lambda i: (i, 0)),
            pl.BlockSpec((1024, 128), lambda i: (i, 0)),
            pl.BlockSpec((1024, 1), lambda i: (i, 0)),
            pl.BlockSpec((256, 128), lambda i: (0, 0)),
            pl.BlockSpec((8, 128), lambda i: (0, 0)),
            pl.BlockSpec((1024, 1), lambda i: (i, 0)),
        ],
        out_specs=pl.BlockSpec((1024, 128), lambda i: (i, 0)),
        out_shape=jax.ShapeDtypeStruct((N, 128), _F32),
    )(hl, hr, inv, W, b, mask)


def _k1_body(src_h, dst_h, delta_h, sp_h, si_h, sj_h, ap_h,
             ks_o, kd_o, ex_o, inv_o, tot_o,
             src2, dst2, nodef, sjf, toti, ranki,
             ksb, kdb, exb, sp_s, tmp, tmpi, tots, zbf, s16, flat256, apv,
             accA, shf, shi, sh16, sem):
    t = lax.axis_index("s")
    sl = pl.ds(t * SL, SL)
    ZV = jnp.zeros((16,), _F32)
    ZI = jnp.zeros((16,), _I32)

    def share_i(slice_ref, full_ref):
        pltpu.sync_copy(slice_ref, shi.at[sl])
        plsc.subcore_barrier()
        pltpu.sync_copy(shi, full_ref)
        plsc.subcore_barrier()

    def share_f(slice_ref, full_ref):
        pltpu.sync_copy(slice_ref, shf.at[sl])
        plsc.subcore_barrier()
        pltpu.sync_copy(shf, full_ref)
        plsc.subcore_barrier()

    def fire_adds(val2, idx2, acc):
        for w in range(NCH // WAVE):
            ds_ = [pltpu.async_copy(val2.at[w * WAVE + b],
                                    acc.at[idx2.at[w * WAVE + b]],
                                    sem, add=True)
                   for b in range(WAVE)]
            for dd in ds_:
                dd.wait()

    pltpu.sync_copy(src_h.at[t], src2)
    pltpu.sync_copy(dst_h.at[t], dst2)
    pltpu.sync_copy(delta_h, nodef)
    pltpu.sync_copy(sp_h.at[sl], sp_s)
    pltpu.sync_copy(sj_h, sjf)
    pltpu.sync_copy(ap_h, apv)

    def zf(j, _):
        tmp[pl.ds(j * 16, 16)] = ZV
        zbf[pl.ds(j * 16, 16)] = ZV
        return 0
    lax.fori_loop(0, SL // 16, zf, 0)
    pltpu.sync_copy(tmp, accA.at[sl])
    plsc.subcore_barrier()

    def av(c, _):
        def iv(j, _):
            idx = pl.ds(j * 16, 16)
            exb[c, idx] = plsc.load_gather(nodef, [src2[c, idx]])
            return 0
        lax.fori_loop(0, EC // 16, iv, 0)
        return 0
    lax.fori_loop(0, NCH, av, 0)
    fire_adds(exb, dst2, accA)
    plsc.subcore_barrier()
    pltpu.sync_copy(accA.at[sl], tmp)
    pltpu.sync_copy(zbf, accA.at[sl])

    def pv(j, anyv):
        idx = pl.ds(j * 16, 16)
        x = sp_s[idx] + tmp[idx]
        p = 1.0 / (1.0 + jnp.exp(-x))
        sp_s[idx] = p
        selv = (p >= TAU).astype(_I32)
        tots[idx] = selv
        tmpi[idx] = selv
        return jnp.maximum(anyv, selv)
    anyv = lax.fori_loop(0, SL // 16, pv, ZI)
    s16[pl.ds(0, 16)] = anyv
    pltpu.sync_copy(s16, sh16.at[pl.ds(t * 16, 16)])
    share_i(tmpi, ranki)
    pltpu.sync_copy(sh16, flat256)

    def rmax(k, a):
        return jnp.maximum(a, flat256[pl.ds(k * 16, 16)])
    any_s = jnp.max(lax.fori_loop(0, 16, rmax, ZI))

    for _hop in range(2):
        def bv(c, _):
            def ivb(j, _):
                idx = pl.ds(j * 16, 16)
                exb[c, idx] = plsc.load_gather(
                    ranki, [dst2[c, idx]]).astype(_F32)
                return 0
            lax.fori_loop(0, EC // 16, ivb, 0)
            return 0
        lax.fori_loop(0, NCH, bv, 0)
        fire_adds(exb, src2, accA)
        plsc.subcore_barrier()
        pltpu.sync_copy(accA.at[sl], tmp)
        pltpu.sync_copy(zbf, accA.at[sl])

        def fv(j, _):
            idx = pl.ds(j * 16, 16)
            fn = (tmp[idx] > 0.0).astype(_I32)
            tots[idx] = tots[idx] | fn
            tmpi[idx] = fn
            return 0
        lax.fori_loop(0, SL // 16, fv, 0)
        share_i(tmpi, ranki)

    def tv(j, cnt):
        idx = pl.ds(j * 16, 16)
        v = jnp.where(any_s > 0, tots[idx], 1)
        tots[idx] = v
        return cnt + jnp.sum(v)
    cnt = lax.fori_loop(0, SL // 16, tv, jnp.int32(0))
    s16[pl.ds(0, 16)] = jnp.full((16,), cnt, _I32)
    pltpu.sync_copy(s16, sh16.at[pl.ds(t * 16, 16)])
    share_i(tots, toti)
    pltpu.sync_copy(sh16, flat256)
    lanes = lax.iota(_I32, 16)
    cnts = plsc.load_gather(flat256, [lanes * 16])
    off = jnp.sum(jnp.where(lanes < t, cnts, 0))

    def rv(j, carry):
        idx = pl.ds(j * 16, 16)
        v = tots[idx]
        tmpi[idx] = plsc.cumsum(v) + (off + carry - 1)
        return carry + jnp.sum(v)
    lax.fori_loop(0, SL // 16, rv, jnp.int32(0))
    share_i(tmpi, ranki)

    share_f(sp_s, nodef)
    apl = apv[pl.ds(0, 16)]

    def tj(j, _):
        idx = pl.ds(j * 16, 16)
        sjf[idx] = sjf[idx] + apl * nodef[idx]
        return 0
    lax.fori_loop(0, NP // 16, tj, 0)
    pltpu.sync_copy(si_h, nodef)

    def dvl(c, _):
        def ivd(j, _):
            idx = pl.ds(j * 16, 16)
            sv = src2[c, idx]
            dv = dst2[c, idx]
            ts = plsc.load_gather(toti, [sv])
            td = plsc.load_gather(toti, [dv])
            em = ts & td
            rs = plsc.load_gather(ranki, [sv])
            rd = plsc.load_gather(ranki, [dv])
            emb = em > 0
            ks0 = jnp.where(emb, rs, 0)
            kd0 = jnp.where(emb, rd, 0)
            tks = plsc.load_gather(toti, [ks0])
            tkd = plsc.load_gather(toti, [kd0])
            vab = (em & tks & tkd) > 0
            ksv = jnp.where(vab, ks0, 0)
            kdv = jnp.where(vab, kd0, 0)
            e = plsc.load_gather(nodef, [kdv]) + plsc.load_gather(sjf, [ksv])
            e = jnp.maximum(e, 0.2 * e)
            exv = jnp.where(vab, jnp.exp(e), 0.0)
            ksb[c, idx] = ksv
            kdb[c, idx] = kdv
            exb[c, idx] = exv
            return 0
        lax.fori_loop(0, EC // 16, ivd, 0)
        return 0
    lax.fori_loop(0, NCH, dvl, 0)
    fire_adds(exb, kdb, accA)
    plsc.subcore_barrier()

    pltpu.sync_copy(ksb, ks_o.at[t])
    pltpu.sync_copy(kdb, kd_o.at[t])
    pltpu.sync_copy(exb, ex_o.at[t])
    pltpu.sync_copy(accA.at[sl], tmp)

    def iv2(j, _):
        idx = pl.ds(j * 16, 16)
        tmp[idx] = 1.0 / (tmp[idx] + 1e-16)
        return 0
    lax.fori_loop(0, SL // 16, iv2, 0)
    pltpu.sync_copy(tmp, inv_o.at[sl])

    def tf2(j, _):
        idx = pl.ds(j * 16, 16)
        tmp[idx] = tots[idx].astype(_F32)
        return 0
    lax.fori_loop(0, SL // 16, tf2, 0)
    pltpu.sync_copy(tmp, tot_o.at[sl])


def _edge_prep(src_r, dst_r, delta, sp, si, sj, apv):
    k1 = pl.kernel(
        _k1_body,
        out_type=[
            jax.ShapeDtypeStruct((NT, NCH, EC), _I32),
            jax.ShapeDtypeStruct((NT, NCH, EC), _I32),
            jax.ShapeDtypeStruct((NT, NCH, EC), _F32),
            jax.ShapeDtypeStruct((NP,), _F32),
            jax.ShapeDtypeStruct((NP,), _F32),
        ],
        mesh=plsc.VectorSubcoreMesh(
            core_axis_name="c", subcore_axis_name="s",
            num_cores=1, num_subcores=NT),
        scratch_types=[
            pltpu.VMEM((NCH, EC), _I32),
            pltpu.VMEM((NCH, EC), _I32),
            pltpu.VMEM((NP,), _F32),
            pltpu.VMEM((NP,), _F32),
            pltpu.VMEM((NP,), _I32),
            pltpu.VMEM((NP,), _I32),
            pltpu.VMEM((NCH, EC), _I32),
            pltpu.VMEM((NCH, EC), _I32),
            pltpu.VMEM((NCH, EC), _F32),
            pltpu.VMEM((SL,), _F32),
            pltpu.VMEM((SL,), _F32),
            pltpu.VMEM((SL,), _I32),
            pltpu.VMEM((SL,), _I32),
            pltpu.VMEM((SL,), _F32),
            pltpu.VMEM((16,), _I32),
            pltpu.VMEM((256,), _I32),
            pltpu.VMEM((16,), _F32),
            pltpu.VMEM_SHARED((NP,), _F32),
            pltpu.VMEM_SHARED((NP,), _F32),
            pltpu.VMEM_SHARED((NP,), _I32),
            pltpu.VMEM_SHARED((256,), _I32),
            pltpu.SemaphoreType.DMA,
        ],
        compiler_params=pltpu.CompilerParams(needs_layout_passes=False),
    )
    return k1(src_r, dst_r, delta, sp, si, sj, apv)


def _conv_body(hxl_h, hxr_h, ks_h, kd_h, ex_h, aggl_h, aggr_h,
               ks0, kd0, ex0, ks1, kd1, ex1, kc0, kc1, rows0, rows1,
               kst, kdt, ext, acc, sg0, sg1, ss0, ss1, si0, si1):
    cid = lax.axis_index("c")
    s = lax.axis_index("s")
    ZV = jnp.zeros((16,), _F32)
    A = (ks0, kd0, ex0, kc0, rows0, sg0, ss0, si0)
    B = (ks1, kd1, ex1, kc1, rows1, sg1, ss1, si1)

    def work(hx_h, agg_h):
        base0 = s * ET

        def fire_idx(c, P):
            ksb, kdb, exb, _, _, _, _, sip = P
            off = base0 + c * EC2
            pltpu.async_copy(ks_h.at[pl.ds(off, EC2)], ksb, sip)
            pltpu.async_copy(kd_h.at[pl.ds(off, EC2)], kdb, sip)
            pltpu.async_copy(ex_h.at[pl.ds(off, EC2)], exb, sip)

        def wait_idx(P):
            ksb, kdb, exb, _, _, _, _, sip = P
            pltpu.make_async_copy(ks_h.at[pl.ds(base0, EC2)], ksb, sip).wait()
            pltpu.make_async_copy(kd_h.at[pl.ds(base0, EC2)], kdb, sip).wait()
            pltpu.make_async_copy(ex_h.at[pl.ds(base0, EC2)], exb, sip).wait()

        def fire_gather(P):
            ksb, _, _, _, rw, sgp, _, _ = P
            pltpu.async_copy(hx_h.at[ksb], rw, sgp)

        def wait_gather(P):
            ksb, _, _, _, rw, sgp, _, _ = P
            pltpu.make_async_copy(hx_h.at[ksb], rw, sgp).wait()

        def fire_scatter(P):
            _, _, _, kcp, rw, _, ssp, _ = P
            pltpu.async_copy(rw, acc.at[kcp], ssp, add=True)

        def wait_scatter(P):
            _, _, _, kcp, rw, _, ssp, _ = P
            pltpu.make_async_copy(rw, acc.at[kcp], ssp).wait()

        def scale(P):
            _, kdb, exb, kcp, rw, _, _, _ = P

            def rowf(rf, _):
                for rr in range(2):
                    r = rf * 2 + rr
                    wv = plsc.load_gather(exb, [jnp.full((16,), r, _I32)])
                    for k in range(8):
                        idx = pl.ds(k * 16, 16)
                        rw[r, idx] = rw[r, idx] * wv
                return 0
            lax.fori_loop(0, EC2 // 2, rowf, 0)
            for j in range(EC2 // 16):
                idx = pl.ds(j * 16, 16)
                kcp[idx] = kdb[idx]

        def zr(r, _):
            for k in range(8):
                rows0[r, pl.ds(k * 16, 16)] = ZV
            return 0
        lax.fori_loop(0, EC2, zr, 0)
        for k in range(SL // EC2):
            pltpu.sync_copy(rows0, acc.at[pl.ds(s * SL + k * EC2, EC2)])
        plsc.subcore_barrier()

        fire_idx(0, A)
        fire_idx(1, B)
        wait_idx(A)
        fire_gather(A)
        NPAIR = NC2 // 2

        def pair(w, _):
            c0 = w * 2

            wait_gather(A)

            @pl.when(w > 0)
            def _():
                wait_scatter(B)
            wait_idx(B)
            fire_gather(B)
            scale(A)
            fire_scatter(A)

            @pl.when(w < NPAIR - 1)
            def _():
                fire_idx(c0 + 2, A)

            wait_gather(B)
            wait_scatter(A)

            @pl.when(w < NPAIR - 1)
            def _():
                wait_idx(A)
                fire_gather(A)
            scale(B)
            fire_scatter(B)

            @pl.when(w < NPAIR - 1)
            def _():
                fire_idx(c0 + 3, B)
            return 0
        lax.fori_loop(0, NPAIR, pair, 0)
        wait_scatter(B)

        toff = base0 + NC2 * EC2
        pltpu.sync_copy(ks_h.at[pl.ds(toff, TAIL)], kst)
        pltpu.sync_copy(kd_h.at[pl.ds(toff, TAIL)], kdt)
        pltpu.sync_copy(ex_h.at[pl.ds(toff, TAIL)], ext)
        pltpu.async_copy(hx_h.at[kst], rows0.at[pl.ds(0, TAIL)], sg0).wait()

        def rowt(r, _):
            wv = plsc.load_gather(ext, [jnp.full((16,), r, _I32)])
            for k in range(8):
                idx = pl.ds(k * 16, 16)
                rows0[r, idx] = rows0[r, idx] * wv
            return 0
        lax.fori_loop(0, TAIL, rowt, 0)
        pltpu.async_copy(rows0.at[pl.ds(0, TAIL)], acc.at[kdt],
                         ss0, add=True).wait()

        plsc.subcore_barrier()
        pltpu.sync_copy(acc.at[pl.ds(s * SL, SL)], agg_h.at[pl.ds(s * SL, SL)])

    @pl.when(cid == 0)
    def _():
        work(hxl_h, aggl_h)

    @pl.when(cid == 1)
    def _():
        work(hxr_h, aggr_h)


def _conv(hxl, hxr, ks, kd, ex):
    k2 = pl.kernel(
        _conv_body,
        out_type=[
            jax.ShapeDtypeStruct((NP, 128), _F32),
            jax.ShapeDtypeStruct((NP, 128), _F32),
        ],
        mesh=plsc.VectorSubcoreMesh(
            core_axis_name="c", subcore_axis_name="s",
            num_cores=2, num_subcores=NT),
        scratch_types=[
            pltpu.VMEM((EC2,), _I32),
            pltpu.VMEM((EC2,), _I32),
            pltpu.VMEM((EC2,), _F32),
            pltpu.VMEM((EC2,), _I32),
            pltpu.VMEM((EC2,), _I32),
            pltpu.VMEM((EC2,), _F32),
            pltpu.VMEM((EC2,), _I32),
            pltpu.VMEM((EC2,), _I32),
            pltpu.VMEM((EC2, 128), _F32),
            pltpu.VMEM((EC2, 128), _F32),
            pltpu.VMEM((TAIL,), _I32),
            pltpu.VMEM((TAIL,), _I32),
            pltpu.VMEM((TAIL,), _F32),
            pltpu.VMEM_SHARED((NP, 128), _F32),
            pltpu.SemaphoreType.DMA,
            pltpu.SemaphoreType.DMA,
            pltpu.SemaphoreType.DMA,
            pltpu.SemaphoreType.DMA,
            pltpu.SemaphoreType.DMA,
            pltpu.SemaphoreType.DMA,
        ],
        compiler_params=pltpu.CompilerParams(needs_layout_passes=False),
    )
    return k2(hxl, hxr, ks, kd, ex)


def kernel(x, edge_index, W_in, b_in, W1, b1, W2, b2, W3, b3,
           W_out, b_out, att_w, att_b, wp):
    src_r = edge_index[0].reshape(NT, NCH, EC)
    dst_r = edge_index[1].reshape(NT, NCH, EC)
    Ws = jnp.pad(jnp.concatenate([wp, att_w[0:256], att_w[256:512]], axis=1),
                 ((0, 0), (0, 125)))
    apv = jnp.full((16,), att_w[512, 0], _F32)

    def b8(b):
        return jnp.broadcast_to(b[None, :], (8, b.shape[0]))

    h1l, h1r, scal = _prelude(x, W_in, b8(b_in), Ws, W1, b8(b1))
    sp = scal[:, 0]
    si = scal[:, 1] + att_b[0]
    sj = scal[:, 2]
    delta = scal[:, 3]

    ks, kd, ex, inv, totf = _edge_prep(src_r, dst_r, delta, sp, si, sj, apv)
    ks = ks.reshape(E)
    kd = kd.reshape(E)
    ex = ex.reshape(E)

    inv2 = inv.reshape(NP, 1)
    a1l, a1r = _conv(h1l, h1r, ks, kd, ex)
    h2l, h2r = _layer(a1l, a1r, inv2, W2, b8(b2))
    a2l, a2r = _conv(h2l, h2r, ks, kd, ex)
    h3l, h3r = _layer(a2l, a2r, inv2, W3, b8(b3))
    a3l, a3r = _conv(h3l, h3r, ks, kd, ex)
    return _final(a3l, a3r, inv2, W_out, b8(b_out), totf.reshape(NP, 1))
